# baseline (speedup 1.0000x reference)
import functools

import jax
import jax.numpy as jnp
import numpy as np
from jax import lax
from jax.experimental import pallas as pl
from jax.experimental.pallas import tpu as pltpu
from jax.experimental.pallas import tpu_sc as plsc

L = 16
NC = 2
NS = 16
C = 2048


def _sc_pool(x, row, col, attr, order, *, N, D, E, EP):
  H = ((N + NC * NS * 128 - 1) // (NC * NS * 128)) * NS * 128
  NP = NC * H
  EC = EP // NS
  NCH = EC // C
  GC = C // L
  TN = H // NS
  KG = TN // L
  DG = D // L
  U = 4

  def body(x_hbm, row_hbm, col_hbm, attr_hbm, order_hbm, out_hbm,
           order_v, cnt_v, rowb, colb, attrb, rowc, colc, attrc,
           selw_v, cnt2, selr, lsel, gb, xb, ab, idxg, idxs, red,
           rsem, csem, asem, zsem,
           stag_sh, selcnt_sh, agg_sh):
    c = lax.axis_index("c")
    s = lax.axis_index("s")
    base_n = c * H
    e0 = pl.multiple_of(s * EC, 128)
    nb0 = pl.multiple_of(s * TN, 128)

    zeros = jnp.zeros((L,), jnp.float32)
    ones = jnp.ones((L,), jnp.float32)
    iota = lax.iota(jnp.int32, L)

    _ns_stage = jax.named_scope("ph_stage"); _ns_stage.__enter__()
    pltpu.sync_copy(order_hbm, order_v)
    for j in range(L):
      for r in range(DG):
        gb[j, pl.ds(r * L, L)] = zeros
    for k in range(KG):
      pltpu.async_copy(
          gb, agg_sh.at[pl.ds(pl.multiple_of(nb0 + k * L, 8), L)], zsem)

    def zc(i, _):
      o = pl.multiple_of(i * (L * 8), L)
      for u in range(8):
        cnt_v[pl.ds(o + u * L, L)] = zeros
      return 0
    lax.fori_loop(0, NP // (L * 8), zc, 0)

    _ns_stage.__exit__(None, None, None)
    _ns = jax.named_scope("ph_pass1"); _ns.__enter__()
    pltpu.async_copy(row_hbm.at[pl.ds(e0, C)], rowb.at[pl.ds(0, C)], rsem)
    pltpu.async_copy(col_hbm.at[pl.ds(e0, C)], colb.at[pl.ds(0, C)], csem)

    def chunk1(ci, _):
      b = pl.multiple_of(lax.rem(ci, 2) * C, 128)
      eo = pl.multiple_of(e0 + ci * C, 128)
      pltpu.make_async_copy(
          row_hbm.at[pl.ds(eo, C)], rowb.at[pl.ds(b, C)], rsem).wait()
      pltpu.make_async_copy(
          col_hbm.at[pl.ds(eo, C)], colb.at[pl.ds(b, C)], csem).wait()

      @pl.when(ci + 1 < NCH)
      def _():
        nb = pl.multiple_of(lax.rem(ci + 1, 2) * C, 128)
        no = pl.multiple_of(e0 + (ci + 1) * C, 128)
        pltpu.async_copy(row_hbm.at[pl.ds(no, C)], rowb.at[pl.ds(nb, C)], rsem)
        pltpu.async_copy(col_hbm.at[pl.ds(no, C)], colb.at[pl.ds(nb, C)], csem)

      def p1_body(g, masked):
        o = pl.multiple_of(b + g * L, L)
        rg = rowb[pl.ds(o, L)]
        cg = colb[pl.ds(o, L)]
        orv = plsc.load_gather(order_v, [rg])
        ocv = plsc.load_gather(order_v, [cg])
        fc = jnp.where(orv <= ocv, ones, zeros)
        fr = jnp.where(ocv <= orv, ones, zeros)
        vmask = ((eo + g * L + iota) < E) if masked else None
        plsc.addupdate_scatter(cnt_v, [cg], fc, mask=vmask)
        plsc.addupdate_scatter(cnt_v, [rg], fr, mask=vmask)

      no_tail = (eo + C) <= E

      @pl.when(no_tail)
      def _():
        plsc.parallel_loop(0, GC, unroll=8)(
            lambda g: p1_body(g, False))

      @pl.when(jnp.logical_not(no_tail))
      def _():
        plsc.parallel_loop(0, GC, unroll=8)(
            lambda g: p1_body(g, True))
      return 0
    lax.fori_loop(0, NCH, chunk1, 0)

    _ns.__exit__(None, None, None)
    _ns = jax.named_scope("ph_reduce"); _ns.__enter__()
    pltpu.sync_copy(cnt_v.at[pl.ds(pl.multiple_of(base_n, 128), H)],
                    stag_sh.at[pl.ds(pl.multiple_of(s * H, 128), H)])
    plsc.subcore_barrier()
    for t in range(NS):
      pltpu.async_copy(
          stag_sh.at[pl.ds(pl.multiple_of(t * H + nb0, 128), TN)],
          cnt2.at[pl.ds(t * TN, TN)], rsem)
    for t in range(NS):
      pltpu.make_async_copy(
          stag_sh.at[pl.ds(pl.multiple_of(t * H + nb0, 128), TN)],
          cnt2.at[pl.ds(t * TN, TN)], rsem).wait()
    def rd(k, _):
      o = pl.multiple_of(k * L, L)
      acc = cnt2[pl.ds(o, L)]
      for t in range(1, NS):
        acc = acc + cnt2[pl.ds(t * TN + o, L)]
      red[pl.ds(o, L)] = acc
      return 0
    lax.fori_loop(0, KG, rd, 0)
    pltpu.sync_copy(red, selcnt_sh.at[pl.ds(nb0, TN)])
    plsc.subcore_barrier()

    pltpu.sync_copy(selcnt_sh, selr)
    def sw(k, _):
      o = pl.multiple_of(k * (L * U), L)
      for u in range(U):
        ou = o + u * L
        selw_v[pl.ds(ou, L)] = jnp.where(selr[pl.ds(ou, L)] == 0.0,
                                         ones, zeros)
      return 0
    lax.fori_loop(0, H // (L * U), sw, 0)

    for k in range(KG):
      pltpu.make_async_copy(
          gb, agg_sh.at[pl.ds(pl.multiple_of(nb0 + k * L, 8), L)], zsem).wait()
    plsc.subcore_barrier()

    _ns.__exit__(None, None, None)
    _ns = jax.named_scope("ph_pass2"); _ns.__enter__()
    pltpu.async_copy(row_hbm.at[pl.ds(e0, C)], rowb.at[pl.ds(0, C)], rsem)
    pltpu.async_copy(col_hbm.at[pl.ds(e0, C)], colb.at[pl.ds(0, C)], csem)
    pltpu.async_copy(attr_hbm.at[pl.ds(e0, C)], attrb.at[pl.ds(0, C)], asem)

    def chunk2(ci, _):
      b = pl.multiple_of(lax.rem(ci, 2) * C, 128)
      eo = pl.multiple_of(e0 + ci * C, 128)
      pltpu.make_async_copy(
          row_hbm.at[pl.ds(eo, C)], rowb.at[pl.ds(b, C)], rsem).wait()
      pltpu.make_async_copy(
          col_hbm.at[pl.ds(eo, C)], colb.at[pl.ds(b, C)], csem).wait()
      pltpu.make_async_copy(
          attr_hbm.at[pl.ds(eo, C)], attrb.at[pl.ds(b, C)], asem).wait()

      @pl.when(ci + 1 < NCH)
      def _():
        nb = pl.multiple_of(lax.rem(ci + 1, 2) * C, 128)
        no = pl.multiple_of(e0 + (ci + 1) * C, 128)
        pltpu.async_copy(row_hbm.at[pl.ds(no, C)], rowb.at[pl.ds(nb, C)], rsem)
        pltpu.async_copy(col_hbm.at[pl.ds(no, C)], colb.at[pl.ds(nb, C)], csem)
        pltpu.async_copy(attr_hbm.at[pl.ds(no, C)], attrb.at[pl.ds(nb, C)],
                         asem)

      def cp_body(g, off, masked):
        o = pl.multiple_of(b + g * L, L)
        rg = rowb[pl.ds(o, L)]
        cg = colb[pl.ds(o, L)]
        ag = attrb[pl.ds(o, L)]
        lc = cg - base_n
        inr = (lc >= 0) & (lc < H)
        lcc = jnp.minimum(jnp.maximum(lc, 0), H - 1)
        selv = plsc.load_gather(selw_v, [lcc])
        keep = inr & (selv > 0.5)
        if masked:
          keep = keep & ((eo + g * L + iota) < E)
        plsc.store_compressed(rowc.at[pl.ds(off, L)], rg, mask=keep)
        plsc.store_compressed(colc.at[pl.ds(off, L)], lcc, mask=keep)
        plsc.store_compressed(attrc.at[pl.ds(off, L)], ag, mask=keep)
        return off + jnp.max(plsc.all_reduce_population_count(keep))

      no_tail = (eo + C) <= E
      kept = lax.cond(
          no_tail,
          lambda: plsc.parallel_loop(0, GC, unroll=4, carry=jnp.int32(0))(
              lambda g, off: cp_body(g, off, False)),
          lambda: plsc.parallel_loop(0, GC, unroll=4, carry=jnp.int32(0))(
              lambda g, off: cp_body(g, off, True)))

      g2n = (kept + (L - 1)) // L

      def p2(g2, _):
        o2 = pl.multiple_of(g2 * L, L)
        valid = (iota + o2) < kept
        rg = jnp.where(valid, rowc[pl.ds(o2, L)], 0)
        lcg = jnp.where(valid, colc[pl.ds(o2, L)], 0)
        idxg[0] = rg
        pltpu.sync_copy(x_hbm.at[idxg.at[0]], gb)
        for j in range(L):
          av = plsc.load_gather(attrc, [jnp.full((L,), o2 + j, jnp.int32)])
          av = av * jnp.where(o2 + j < kept, 1.0, 0.0)
          for r in range(DG):
            gb[j, pl.ds(r * L, L)] = gb[j, pl.ds(r * L, L)] * av
        idxs[0] = lcg
        pltpu.sync_copy(gb, agg_sh.at[idxs.at[0]], add=True)
        return 0
      lax.fori_loop(0, g2n, p2, 0)
      return 0
    lax.fori_loop(0, NCH, chunk2, 0)

    _ns.__exit__(None, None, None)
    _ns = jax.named_scope("ph_epi"); _ns.__enter__()
    plsc.subcore_barrier()

    for j in range(L):
      for r in range(DG):
        gb[j, pl.ds(r * L, L)] = zeros
    def zf(k, _):
      st = pl.multiple_of(base_n, 8) + pl.multiple_of(nb0 + k * L, 8)
      @pl.when(st < N)
      def _():
        pltpu.async_copy(gb, out_hbm.at[pl.ds(st, L)], zsem)
      return 0
    lax.fori_loop(0, KG, zf, 0)

    def sel_cp(k, off):
      lo = pl.multiple_of(nb0 + k * L, 8)
      ids = lo + iota
      m = (selw_v[pl.ds(lo, L)] > 0.5) & ((base_n + ids) < N)
      plsc.store_compressed(lsel.at[pl.ds(off, L)], ids, mask=m)
      return off + jnp.max(plsc.all_reduce_population_count(m))
    scnt = lax.fori_loop(0, KG, sel_cp, jnp.int32(0))

    def zd(k, _):
      st = pl.multiple_of(base_n, 8) + pl.multiple_of(nb0 + k * L, 8)
      @pl.when(st < N)
      def _():
        pltpu.make_async_copy(gb, out_hbm.at[pl.ds(st, L)], zsem).wait()
      return 0
    lax.fori_loop(0, KG, zd, 0)

    g3n = (scnt + (L - 1)) // L

    def p3(g3, _):
      o3 = pl.multiple_of(g3 * L, L)
      valid = (iota + o3) < scnt
      last = plsc.load_gather(lsel, [jnp.full((L,), scnt - 1, jnp.int32)])
      ids = jnp.where(valid, lsel[pl.ds(o3, L)], last)
      gids = ids + base_n
      idxg[0] = gids
      pltpu.sync_copy(x_hbm.at[idxg.at[0]], xb)
      idxs[0] = ids
      pltpu.sync_copy(agg_sh.at[idxs.at[0]], ab)
      for j in range(L):
        for r in range(DG):
          xb[j, pl.ds(r * L, L)] = xb[j, pl.ds(r * L, L)] + ab[j, pl.ds(r * L, L)]
      pltpu.sync_copy(xb, out_hbm.at[idxg.at[0]])
      return 0
    lax.fori_loop(0, g3n, p3, 0)
    _ns.__exit__(None, None, None)

  mesh = plsc.VectorSubcoreMesh(
      core_axis_name="c", subcore_axis_name="s", num_cores=NC, num_subcores=NS)
  run = pl.kernel(
      body,
      out_type=jax.ShapeDtypeStruct((N, D), jnp.float32),
      mesh=mesh,
      compiler_params=pltpu.CompilerParams(needs_layout_passes=False),
      scratch_types=[
          pltpu.VMEM((N,), jnp.float32),
          pltpu.VMEM((NP,), jnp.float32),
          pltpu.VMEM((2 * C,), jnp.int32),
          pltpu.VMEM((2 * C,), jnp.int32),
          pltpu.VMEM((2 * C,), jnp.float32),
          pltpu.VMEM((C,), jnp.int32),
          pltpu.VMEM((C,), jnp.int32),
          pltpu.VMEM((C,), jnp.float32),
          pltpu.VMEM((H,), jnp.float32),
          pltpu.VMEM((NS * TN,), jnp.float32),
          pltpu.VMEM((H,), jnp.float32),
          pltpu.VMEM((TN,), jnp.int32),
          pltpu.VMEM((L, D), jnp.float32),
          pltpu.VMEM((L, D), jnp.float32),
          pltpu.VMEM((L, D), jnp.float32),
          pltpu.VMEM((1, L), jnp.int32),
          pltpu.VMEM((1, L), jnp.int32),
          pltpu.VMEM((TN,), jnp.float32),
          pltpu.SemaphoreType.DMA,
          pltpu.SemaphoreType.DMA,
          pltpu.SemaphoreType.DMA,
          pltpu.SemaphoreType.DMA,
          pltpu.VMEM_SHARED((NS * H,), jnp.float32),
          pltpu.VMEM_SHARED((H,), jnp.float32),
          pltpu.VMEM_SHARED((H, D), jnp.float32),
      ],
  )
  return run(x, row, col, attr, order)


@jax.jit
def kernel(x, edge_index, edge_attr, batch):
  N, D = x.shape
  E = edge_index.shape[1]
  assert D % L == 0 and N % L == 0
  order = jax.random.uniform(jax.random.key(42), (N,), dtype=jnp.float32)
  EP = ((E + NS * C - 1) // (NS * C)) * (NS * C)
  pad = EP - E
  row = jnp.concatenate([edge_index[0], jnp.zeros((pad,), jnp.int32)])
  col = jnp.concatenate([edge_index[1], jnp.zeros((pad,), jnp.int32)])
  attr = jnp.concatenate([edge_attr, jnp.zeros((pad,), jnp.float32)])
  return _sc_pool(x, row, col, attr, order, N=N, D=D, E=E, EP=EP)


# speedup vs baseline: 1.8492x; 1.8492x over previous
import functools

import jax
import jax.numpy as jnp
import numpy as np
from jax import lax
from jax.experimental import pallas as pl
from jax.experimental.pallas import tpu as pltpu
from jax.experimental.pallas import tpu_sc as plsc

L = 16
NC = 2
NS = 16
C = 2048
BE = 64


def _sc_pool(x, row, col, attr, order, *, N, D, E, EP):
  H = ((N + NC * NS * 128 - 1) // (NC * NS * 128)) * NS * 128
  NP = NC * H
  EC = EP // NS
  NCH = EC // C
  GC = C // L
  TN = H // NS
  KG = TN // L
  DG = D // L
  U = 4

  def body(x_hbm, row_hbm, col_hbm, attr_hbm, order_hbm, out_hbm,
           order_v, cnt_v, rowb, colb, attrb, rowc, colc, attrc,
           selw_v, cnt2, selr, lsel, gb, xb, ab, idxg, idxs,
           idxg16, idxs16, red, rsem, csem, asem, zsem,
           stag_sh, selcnt_sh, agg_sh):
    c = lax.axis_index("c")
    s = lax.axis_index("s")
    base_n = c * H
    e0 = pl.multiple_of(s * EC, 128)
    nb0 = pl.multiple_of(s * TN, 128)

    zeros = jnp.zeros((L,), jnp.float32)
    ones = jnp.ones((L,), jnp.float32)
    iota = lax.iota(jnp.int32, L)

    _ns_stage = jax.named_scope("ph_stage"); _ns_stage.__enter__()
    pltpu.sync_copy(order_hbm, order_v)
    for j in range(L):
      for r in range(DG):
        gb[j, pl.ds(r * L, L)] = zeros
    for k in range(KG):
      pltpu.async_copy(
          gb.at[pl.ds(0, L)],
          agg_sh.at[pl.ds(pl.multiple_of(nb0 + k * L, 8), L)], zsem)

    def zc(i, _):
      o = pl.multiple_of(i * (L * 8), L)
      for u in range(8):
        cnt_v[pl.ds(o + u * L, L)] = zeros
      return 0
    lax.fori_loop(0, NP // (L * 8), zc, 0)

    _ns_stage.__exit__(None, None, None)
    _ns = jax.named_scope("ph_pass1"); _ns.__enter__()
    pltpu.async_copy(row_hbm.at[pl.ds(e0, C)], rowb.at[pl.ds(0, C)], rsem)
    pltpu.async_copy(col_hbm.at[pl.ds(e0, C)], colb.at[pl.ds(0, C)], csem)

    def chunk1(ci, _):
      b = pl.multiple_of(lax.rem(ci, 2) * C, 128)
      eo = pl.multiple_of(e0 + ci * C, 128)
      pltpu.make_async_copy(
          row_hbm.at[pl.ds(eo, C)], rowb.at[pl.ds(b, C)], rsem).wait()
      pltpu.make_async_copy(
          col_hbm.at[pl.ds(eo, C)], colb.at[pl.ds(b, C)], csem).wait()

      @pl.when(ci + 1 < NCH)
      def _():
        nb = pl.multiple_of(lax.rem(ci + 1, 2) * C, 128)
        no = pl.multiple_of(e0 + (ci + 1) * C, 128)
        pltpu.async_copy(row_hbm.at[pl.ds(no, C)], rowb.at[pl.ds(nb, C)], rsem)
        pltpu.async_copy(col_hbm.at[pl.ds(no, C)], colb.at[pl.ds(nb, C)], csem)

      def p1_body(g, masked):
        o = pl.multiple_of(b + g * L, L)
        rg = rowb[pl.ds(o, L)]
        cg = colb[pl.ds(o, L)]
        orv = plsc.load_gather(order_v, [rg])
        ocv = plsc.load_gather(order_v, [cg])
        fc = jnp.where(orv <= ocv, ones, zeros)
        fr = jnp.where(ocv <= orv, ones, zeros)
        vmask = ((eo + g * L + iota) < E) if masked else None
        plsc.addupdate_scatter(cnt_v, [cg], fc, mask=vmask)
        plsc.addupdate_scatter(cnt_v, [rg], fr, mask=vmask)

      no_tail = (eo + C) <= E

      @pl.when(no_tail)
      def _():
        plsc.parallel_loop(0, GC, unroll=8)(
            lambda g: p1_body(g, False))

      @pl.when(jnp.logical_not(no_tail))
      def _():
        plsc.parallel_loop(0, GC, unroll=8)(
            lambda g: p1_body(g, True))
      return 0
    lax.fori_loop(0, NCH, chunk1, 0)

    _ns.__exit__(None, None, None)
    _ns = jax.named_scope("ph_reduce"); _ns.__enter__()
    pltpu.sync_copy(cnt_v.at[pl.ds(pl.multiple_of(base_n, 128), H)],
                    stag_sh.at[pl.ds(pl.multiple_of(s * H, 128), H)])
    plsc.subcore_barrier()
    for t in range(NS):
      pltpu.async_copy(
          stag_sh.at[pl.ds(pl.multiple_of(t * H + nb0, 128), TN)],
          cnt2.at[pl.ds(t * TN, TN)], rsem)
    for t in range(NS):
      pltpu.make_async_copy(
          stag_sh.at[pl.ds(pl.multiple_of(t * H + nb0, 128), TN)],
          cnt2.at[pl.ds(t * TN, TN)], rsem).wait()
    def rd(k, _):
      o = pl.multiple_of(k * L, L)
      acc = cnt2[pl.ds(o, L)]
      for t in range(1, NS):
        acc = acc + cnt2[pl.ds(t * TN + o, L)]
      red[pl.ds(o, L)] = acc
      return 0
    lax.fori_loop(0, KG, rd, 0)
    pltpu.sync_copy(red, selcnt_sh.at[pl.ds(nb0, TN)])
    plsc.subcore_barrier()

    pltpu.sync_copy(selcnt_sh, selr)
    def sw(k, _):
      o = pl.multiple_of(k * (L * U), L)
      for u in range(U):
        ou = o + u * L
        selw_v[pl.ds(ou, L)] = jnp.where(selr[pl.ds(ou, L)] == 0.0,
                                         ones, zeros)
      return 0
    lax.fori_loop(0, H // (L * U), sw, 0)

    for k in range(KG):
      pltpu.make_async_copy(
          gb.at[pl.ds(0, L)],
          agg_sh.at[pl.ds(pl.multiple_of(nb0 + k * L, 8), L)], zsem).wait()
    plsc.subcore_barrier()

    _ns.__exit__(None, None, None)
    _ns = jax.named_scope("ph_pass2"); _ns.__enter__()
    pltpu.async_copy(row_hbm.at[pl.ds(e0, C)], rowb.at[pl.ds(0, C)], rsem)
    pltpu.async_copy(col_hbm.at[pl.ds(e0, C)], colb.at[pl.ds(0, C)], csem)
    pltpu.async_copy(attr_hbm.at[pl.ds(e0, C)], attrb.at[pl.ds(0, C)], asem)

    def chunk2(ci, rem):
      b = pl.multiple_of(lax.rem(ci, 2) * C, 128)
      eo = pl.multiple_of(e0 + ci * C, 128)
      pltpu.make_async_copy(
          row_hbm.at[pl.ds(eo, C)], rowb.at[pl.ds(b, C)], rsem).wait()
      pltpu.make_async_copy(
          col_hbm.at[pl.ds(eo, C)], colb.at[pl.ds(b, C)], csem).wait()
      pltpu.make_async_copy(
          attr_hbm.at[pl.ds(eo, C)], attrb.at[pl.ds(b, C)], asem).wait()

      @pl.when(ci + 1 < NCH)
      def _():
        nb = pl.multiple_of(lax.rem(ci + 1, 2) * C, 128)
        no = pl.multiple_of(e0 + (ci + 1) * C, 128)
        pltpu.async_copy(row_hbm.at[pl.ds(no, C)], rowb.at[pl.ds(nb, C)], rsem)
        pltpu.async_copy(col_hbm.at[pl.ds(no, C)], colb.at[pl.ds(nb, C)], csem)
        pltpu.async_copy(attr_hbm.at[pl.ds(no, C)], attrb.at[pl.ds(nb, C)],
                         asem)

      def cp_body(g, off, masked):
        o = pl.multiple_of(b + g * L, L)
        rg = rowb[pl.ds(o, L)]
        cg = colb[pl.ds(o, L)]
        ag = attrb[pl.ds(o, L)]
        lc = cg - base_n
        inr = (lc >= 0) & (lc < H)
        lcc = jnp.minimum(jnp.maximum(lc, 0), H - 1)
        selv = plsc.load_gather(selw_v, [lcc])
        keep = inr & (selv > 0.5)
        if masked:
          keep = keep & ((eo + g * L + iota) < E)
        plsc.store_compressed(rowc.at[pl.ds(off, L)], rg, mask=keep)
        plsc.store_compressed(colc.at[pl.ds(off, L)], lcc, mask=keep)
        plsc.store_compressed(attrc.at[pl.ds(off, L)], ag, mask=keep)
        return off + jnp.max(plsc.all_reduce_population_count(keep))

      no_tail = (eo + C) <= E
      kept = lax.cond(
          no_tail,
          lambda: plsc.parallel_loop(0, GC, unroll=4, carry=rem)(
              lambda g, off: cp_body(g, off, False)),
          lambda: plsc.parallel_loop(0, GC, unroll=4, carry=rem)(
              lambda g, off: cp_body(g, off, True)))

      nfull = kept // BE

      def batch(g4, _):
        o4 = pl.multiple_of(g4 * BE, L)
        for k in range(BE // L):
          idxg[0, pl.ds(k * L, L)] = rowc[pl.ds(o4 + k * L, L)]
          idxs[0, pl.ds(k * L, L)] = colc[pl.ds(o4 + k * L, L)]
        pltpu.sync_copy(x_hbm.at[idxg.at[0]], gb)
        for j in range(BE):
          av = plsc.load_gather(attrc, [jnp.full((L,), o4 + j, jnp.int32)])
          for r in range(DG):
            gb[j, pl.ds(r * L, L)] = gb[j, pl.ds(r * L, L)] * av
        pltpu.sync_copy(gb, agg_sh.at[idxs.at[0]], add=True)
        return 0
      lax.fori_loop(0, nfull, batch, 0)

      rem_new = kept - nfull * BE
      for k in range(BE // L):
        @pl.when(k * L < rem_new)
        def _():
          src = pl.multiple_of(nfull * BE, L) + k * L
          rowc[pl.ds(k * L, L)] = rowc[pl.ds(src, L)]
          colc[pl.ds(k * L, L)] = colc[pl.ds(src, L)]
          attrc[pl.ds(k * L, L)] = attrc[pl.ds(src, L)]
      return rem_new
    rem_f = lax.fori_loop(0, NCH, chunk2, jnp.int32(0))

    g2n = (rem_f + (L - 1)) // L

    def p2f(g2, _):
      o2 = pl.multiple_of(g2 * L, L)
      valid = (iota + o2) < rem_f
      rg = jnp.where(valid, rowc[pl.ds(o2, L)], 0)
      lcg = jnp.where(valid, colc[pl.ds(o2, L)], 0)
      idxg16[0] = rg
      pltpu.sync_copy(x_hbm.at[idxg16.at[0]], gb.at[pl.ds(0, L)])
      for j in range(L):
        av = plsc.load_gather(attrc, [jnp.full((L,), o2 + j, jnp.int32)])
        av = av * jnp.where(o2 + j < rem_f, 1.0, 0.0)
        for r in range(DG):
          gb[j, pl.ds(r * L, L)] = gb[j, pl.ds(r * L, L)] * av
      idxs16[0] = lcg
      pltpu.sync_copy(gb.at[pl.ds(0, L)], agg_sh.at[idxs16.at[0]], add=True)
      return 0
    lax.fori_loop(0, g2n, p2f, 0)

    _ns.__exit__(None, None, None)
    _ns = jax.named_scope("ph_epi"); _ns.__enter__()
    plsc.subcore_barrier()

    for j in range(L):
      for r in range(DG):
        gb[j, pl.ds(r * L, L)] = zeros
    def zf(k, _):
      st = pl.multiple_of(base_n, 8) + pl.multiple_of(nb0 + k * L, 8)
      @pl.when(st < N)
      def _():
        pltpu.async_copy(gb.at[pl.ds(0, L)], out_hbm.at[pl.ds(st, L)], zsem)
      return 0
    lax.fori_loop(0, KG, zf, 0)

    def sel_cp(k, off):
      lo = pl.multiple_of(nb0 + k * L, 8)
      ids = lo + iota
      m = (selw_v[pl.ds(lo, L)] > 0.5) & ((base_n + ids) < N)
      plsc.store_compressed(lsel.at[pl.ds(off, L)], ids, mask=m)
      return off + jnp.max(plsc.all_reduce_population_count(m))
    scnt = lax.fori_loop(0, KG, sel_cp, jnp.int32(0))

    def zd(k, _):
      st = pl.multiple_of(base_n, 8) + pl.multiple_of(nb0 + k * L, 8)
      @pl.when(st < N)
      def _():
        pltpu.make_async_copy(
            gb.at[pl.ds(0, L)], out_hbm.at[pl.ds(st, L)], zsem).wait()
      return 0
    lax.fori_loop(0, KG, zd, 0)

    g3n = (scnt + (L - 1)) // L

    def p3(g3, _):
      o3 = pl.multiple_of(g3 * L, L)
      valid = (iota + o3) < scnt
      last = plsc.load_gather(lsel, [jnp.full((L,), scnt - 1, jnp.int32)])
      ids = jnp.where(valid, lsel[pl.ds(o3, L)], last)
      gids = ids + base_n
      idxg16[0] = gids
      idxs16[0] = ids
      pltpu.async_copy(x_hbm.at[idxg16.at[0]], xb, rsem)
      pltpu.async_copy(agg_sh.at[idxs16.at[0]], ab, csem)
      pltpu.make_async_copy(x_hbm.at[idxg16.at[0]], xb, rsem).wait()
      pltpu.make_async_copy(agg_sh.at[idxs16.at[0]], ab, csem).wait()
      for j in range(L):
        for r in range(DG):
          xb[j, pl.ds(r * L, L)] = xb[j, pl.ds(r * L, L)] + ab[j, pl.ds(r * L, L)]
      pltpu.sync_copy(xb, out_hbm.at[idxg16.at[0]])
      return 0
    lax.fori_loop(0, g3n, p3, 0)
    _ns.__exit__(None, None, None)

  mesh = plsc.VectorSubcoreMesh(
      core_axis_name="c", subcore_axis_name="s", num_cores=NC, num_subcores=NS)
  run = pl.kernel(
      body,
      out_type=jax.ShapeDtypeStruct((N, D), jnp.float32),
      mesh=mesh,
      compiler_params=pltpu.CompilerParams(needs_layout_passes=False),
      scratch_types=[
          pltpu.VMEM((N,), jnp.float32),
          pltpu.VMEM((NP,), jnp.float32),
          pltpu.VMEM((2 * C,), jnp.int32),
          pltpu.VMEM((2 * C,), jnp.int32),
          pltpu.VMEM((2 * C,), jnp.float32),
          pltpu.VMEM((C + BE,), jnp.int32),
          pltpu.VMEM((C + BE,), jnp.int32),
          pltpu.VMEM((C + BE,), jnp.float32),
          pltpu.VMEM((H,), jnp.float32),
          pltpu.VMEM((NS * TN,), jnp.float32),
          pltpu.VMEM((H,), jnp.float32),
          pltpu.VMEM((TN,), jnp.int32),
          pltpu.VMEM((BE, D), jnp.float32),
          pltpu.VMEM((L, D), jnp.float32),
          pltpu.VMEM((L, D), jnp.float32),
          pltpu.VMEM((1, BE), jnp.int32),
          pltpu.VMEM((1, BE), jnp.int32),
          pltpu.VMEM((1, L), jnp.int32),
          pltpu.VMEM((1, L), jnp.int32),
          pltpu.VMEM((TN,), jnp.float32),
          pltpu.SemaphoreType.DMA,
          pltpu.SemaphoreType.DMA,
          pltpu.SemaphoreType.DMA,
          pltpu.SemaphoreType.DMA,
          pltpu.VMEM_SHARED((NS * H,), jnp.float32),
          pltpu.VMEM_SHARED((H,), jnp.float32),
          pltpu.VMEM_SHARED((H, D), jnp.float32),
      ],
  )
  return run(x, row, col, attr, order)


@jax.jit
def kernel(x, edge_index, edge_attr, batch):
  N, D = x.shape
  E = edge_index.shape[1]
  assert D % L == 0 and N % L == 0
  order = jax.random.uniform(jax.random.key(42), (N,), dtype=jnp.float32)
  EP = ((E + NS * C - 1) // (NS * C)) * (NS * C)
  pad = EP - E
  row = jnp.concatenate([edge_index[0], jnp.zeros((pad,), jnp.int32)])
  col = jnp.concatenate([edge_index[1], jnp.zeros((pad,), jnp.int32)])
  attr = jnp.concatenate([edge_attr, jnp.zeros((pad,), jnp.float32)])
  return _sc_pool(x, row, col, attr, order, N=N, D=D, E=E, EP=EP)


# speedup vs baseline: 1.9809x; 1.0712x over previous
import functools

import jax
import jax.numpy as jnp
from jax import lax
from jax.experimental import pallas as pl
from jax.experimental.pallas import tpu as pltpu
from jax.experimental.pallas import tpu_sc as plsc

L = 16
NC = 2
NS = 16
C = 2048
BE = 64


def _sc_pool(x, row, col, attr, order, *, N, D, E):
  H = ((N + NC * NS * 128 - 1) // (NC * NS * 128)) * NS * 128
  NP = NC * H
  EC = E // NS
  NCHF = EC // C
  CL = EC - NCHF * C
  NCH = NCHF + (1 if CL else 0)
  TN = H // NS
  KG = TN // L
  DG = D // L

  def clen(ci):
    return C if ci < NCHF else CL

  def body(x_hbm, row_hbm, col_hbm, attr_hbm, order_hbm, out_hbm,
           order_v, cnt_v, rowb, colb, attrb, rowc, colc, attrc,
           selw_v, cnt2, selr, lsel, gb, xb, ab, idxg, idxs,
           idxg16, idxs16, red, rsem, csem, asem, zsem, osem,
           trsem, tcsem, tasem,
           stag_sh, selcnt_sh, agg_sh):
    c = lax.axis_index("c")
    s = lax.axis_index("s")
    base_n = c * H
    e0 = pl.multiple_of(s * EC, 8)
    nb0 = pl.multiple_of(s * TN, 128)

    zeros = jnp.zeros((L,), jnp.float32)
    ones = jnp.ones((L,), jnp.float32)
    iota = lax.iota(jnp.int32, L)

    pltpu.async_copy(order_hbm, order_v, rsem)
    for j in range(L):
      for r in range(DG):
        gb[j, pl.ds(r * L, L)] = zeros
        xb[j, pl.ds(r * L, L)] = zeros
    for k in range(KG):
      pltpu.async_copy(
          gb.at[pl.ds(0, L)],
          agg_sh.at[pl.ds(pl.multiple_of(nb0 + k * L, 8), L)], zsem)

    def zf(k, _):
      st = pl.multiple_of(base_n, 8) + pl.multiple_of(nb0 + k * L, 8)
      @pl.when(st < N)
      def _():
        pltpu.async_copy(xb, out_hbm.at[pl.ds(st, L)], osem)
      return 0
    lax.fori_loop(0, KG, zf, 0)

    def zc(i, _):
      o = pl.multiple_of(i * (L * 8), L)
      for u in range(8):
        cnt_v[pl.ds(o + u * L, L)] = zeros
      return 0
    lax.fori_loop(0, NP // (L * 8), zc, 0)
    pltpu.make_async_copy(order_hbm, order_v, rsem).wait()

    pltpu.async_copy(row_hbm.at[pl.ds(e0, C)], rowb.at[pl.ds(0, C)], rsem)
    pltpu.async_copy(col_hbm.at[pl.ds(e0, C)], colb.at[pl.ds(0, C)], csem)

    def p1_body(b):
      def go(g):
        o = pl.multiple_of(b + g * L, L)
        rg = rowb[pl.ds(o, L)]
        cg = colb[pl.ds(o, L)]
        orv = plsc.load_gather(order_v, [rg])
        ocv = plsc.load_gather(order_v, [cg])
        fc = jnp.where(orv <= ocv, ones, zeros)
        fr = jnp.where(ocv <= orv, ones, zeros)
        plsc.addupdate_scatter(cnt_v, [cg], fc)
        plsc.addupdate_scatter(cnt_v, [rg], fr)
      return go

    if CL:
      eot = pl.multiple_of(e0 + NCHF * C, 8)
      pltpu.async_copy(row_hbm.at[pl.ds(eot, CL)], rowb.at[pl.ds(C, CL)],
                       trsem)
      pltpu.async_copy(col_hbm.at[pl.ds(eot, CL)], colb.at[pl.ds(C, CL)],
                       tcsem)
      pltpu.make_async_copy(
          row_hbm.at[pl.ds(eot, CL)], rowb.at[pl.ds(C, CL)], trsem).wait()
      pltpu.make_async_copy(
          col_hbm.at[pl.ds(eot, CL)], colb.at[pl.ds(C, CL)], tcsem).wait()
      plsc.parallel_loop(0, CL // L, unroll=8)(p1_body(C))

    def chunk1(ci, _):
      b = pl.multiple_of(lax.rem(ci, 2) * C, 128)
      eo = pl.multiple_of(e0 + ci * C, 8)
      pltpu.make_async_copy(
          row_hbm.at[pl.ds(eo, C)], rowb.at[pl.ds(b, C)], rsem).wait()
      pltpu.make_async_copy(
          col_hbm.at[pl.ds(eo, C)], colb.at[pl.ds(b, C)], csem).wait()

      @pl.when(ci + 1 < NCHF)
      def _():
        nb = pl.multiple_of(lax.rem(ci + 1, 2) * C, 128)
        no = pl.multiple_of(e0 + (ci + 1) * C, 8)
        pltpu.async_copy(row_hbm.at[pl.ds(no, C)], rowb.at[pl.ds(nb, C)],
                         rsem)
        pltpu.async_copy(col_hbm.at[pl.ds(no, C)], colb.at[pl.ds(nb, C)],
                         csem)
      plsc.parallel_loop(0, C // L, unroll=8)(p1_body(b))
      return 0
    lax.fori_loop(0, NCHF, chunk1, 0)

    pltpu.sync_copy(cnt_v.at[pl.ds(pl.multiple_of(base_n, 128), H)],
                    stag_sh.at[pl.ds(pl.multiple_of(s * H, 128), H)])
    plsc.subcore_barrier()
    for t in range(NS):
      pltpu.async_copy(
          stag_sh.at[pl.ds(pl.multiple_of(t * H + nb0, 128), TN)],
          cnt2.at[pl.ds(t * TN, TN)], rsem)
    for t in range(NS):
      pltpu.make_async_copy(
          stag_sh.at[pl.ds(pl.multiple_of(t * H + nb0, 128), TN)],
          cnt2.at[pl.ds(t * TN, TN)], rsem).wait()
    def rd(k, _):
      o = pl.multiple_of(k * L, L)
      acc = cnt2[pl.ds(o, L)]
      for t in range(1, NS):
        acc = acc + cnt2[pl.ds(t * TN + o, L)]
      red[pl.ds(o, L)] = acc
      return 0
    lax.fori_loop(0, KG, rd, 0)
    pltpu.sync_copy(red, selcnt_sh.at[pl.ds(nb0, TN)])
    plsc.subcore_barrier()

    pltpu.sync_copy(selcnt_sh, selr)
    def sw(k, _):
      o = pl.multiple_of(k * (L * 4), L)
      for u in range(4):
        ou = o + u * L
        selw_v[pl.ds(ou, L)] = jnp.where(selr[pl.ds(ou, L)] == 0.0,
                                         ones, zeros)
      return 0
    lax.fori_loop(0, H // (L * 4), sw, 0)

    for k in range(KG):
      pltpu.make_async_copy(
          gb.at[pl.ds(0, L)],
          agg_sh.at[pl.ds(pl.multiple_of(nb0 + k * L, 8), L)], zsem).wait()
    plsc.subcore_barrier()

    pltpu.async_copy(row_hbm.at[pl.ds(e0, C)], rowb.at[pl.ds(0, C)], rsem)
    pltpu.async_copy(col_hbm.at[pl.ds(e0, C)], colb.at[pl.ds(0, C)], csem)
    pltpu.async_copy(attr_hbm.at[pl.ds(e0, C)], attrb.at[pl.ds(0, C)], asem)
    if CL:
      eot0 = pl.multiple_of(e0 + NCHF * C, 8)
      pltpu.async_copy(row_hbm.at[pl.ds(eot0, CL)], rowb.at[pl.ds(C, CL)],
                       trsem)
      pltpu.async_copy(col_hbm.at[pl.ds(eot0, CL)], colb.at[pl.ds(C, CL)],
                       tcsem)
      pltpu.async_copy(attr_hbm.at[pl.ds(eot0, CL)], attrb.at[pl.ds(C, CL)],
                       tasem)

    def cp_body(b):
      def go(g, off):
        o = pl.multiple_of(b + g * L, L)
        rg = rowb[pl.ds(o, L)]
        cg = colb[pl.ds(o, L)]
        ag = attrb[pl.ds(o, L)]
        lc = cg - base_n
        inr = (lc >= 0) & (lc < H)
        lcc = jnp.minimum(jnp.maximum(lc, 0), H - 1)
        selv = plsc.load_gather(selw_v, [lcc])
        keep = inr & (selv > 0.5)
        plsc.store_compressed(rowc.at[pl.ds(off, L)], rg, mask=keep)
        plsc.store_compressed(colc.at[pl.ds(off, L)], lcc, mask=keep)
        plsc.store_compressed(attrc.at[pl.ds(off, L)], ag, mask=keep)
        return off + jnp.max(plsc.all_reduce_population_count(keep))
      return go

    def batch(g4, _):
      o4 = pl.multiple_of(g4 * BE, L)
      for k in range(BE // L):
        idxg[0, pl.ds(k * L, L)] = rowc[pl.ds(o4 + k * L, L)]
        idxs[0, pl.ds(k * L, L)] = colc[pl.ds(o4 + k * L, L)]
      pltpu.sync_copy(x_hbm.at[idxg.at[0]], gb)
      def scale(j, _):
        av = plsc.load_gather(attrc, [jnp.full((L,), o4 + j, jnp.int32)])
        for r in range(DG):
          gb[j, pl.ds(r * L, L)] = gb[j, pl.ds(r * L, L)] * av
        return 0
      lax.fori_loop(0, BE, scale, 0)
      pltpu.sync_copy(gb, agg_sh.at[idxs.at[0]], add=True)
      return 0

    def process(kept):
      nfull = kept // BE
      lax.fori_loop(0, nfull, batch, 0)
      rem = kept - nfull * BE
      for k in range(BE // L):
        @pl.when(k * L < rem)
        def _():
          src = pl.multiple_of(nfull * BE, L) + k * L
          rowc[pl.ds(k * L, L)] = rowc[pl.ds(src, L)]
          colc[pl.ds(k * L, L)] = colc[pl.ds(src, L)]
          attrc[pl.ds(k * L, L)] = attrc[pl.ds(src, L)]
      return rem

    rem0 = jnp.int32(0)
    if CL:
      eot = pl.multiple_of(e0 + NCHF * C, 8)
      pltpu.make_async_copy(
          row_hbm.at[pl.ds(eot, CL)], rowb.at[pl.ds(C, CL)], trsem).wait()
      pltpu.make_async_copy(
          col_hbm.at[pl.ds(eot, CL)], colb.at[pl.ds(C, CL)], tcsem).wait()
      pltpu.make_async_copy(
          attr_hbm.at[pl.ds(eot, CL)], attrb.at[pl.ds(C, CL)], tasem).wait()
      kt = plsc.parallel_loop(0, CL // L, unroll=8, carry=rem0)(cp_body(C))
      rem0 = process(kt)

    def chunk2(ci, rem):
      b = pl.multiple_of(lax.rem(ci, 2) * C, 128)
      eo = pl.multiple_of(e0 + ci * C, 8)
      pltpu.make_async_copy(
          row_hbm.at[pl.ds(eo, C)], rowb.at[pl.ds(b, C)], rsem).wait()
      pltpu.make_async_copy(
          col_hbm.at[pl.ds(eo, C)], colb.at[pl.ds(b, C)], csem).wait()
      pltpu.make_async_copy(
          attr_hbm.at[pl.ds(eo, C)], attrb.at[pl.ds(b, C)], asem).wait()

      @pl.when(ci + 1 < NCHF)
      def _():
        nb = pl.multiple_of(lax.rem(ci + 1, 2) * C, 128)
        no = pl.multiple_of(e0 + (ci + 1) * C, 8)
        pltpu.async_copy(row_hbm.at[pl.ds(no, C)], rowb.at[pl.ds(nb, C)],
                         rsem)
        pltpu.async_copy(col_hbm.at[pl.ds(no, C)], colb.at[pl.ds(nb, C)],
                         csem)
        pltpu.async_copy(attr_hbm.at[pl.ds(no, C)], attrb.at[pl.ds(nb, C)],
                         asem)
      kept = plsc.parallel_loop(0, C // L, unroll=8, carry=rem)(cp_body(b))
      return process(kept)
    rem = lax.fori_loop(0, NCHF, chunk2, rem0)

    g2n = (rem + (L - 1)) // L

    def p2f(g2, _):
      o2 = pl.multiple_of(g2 * L, L)
      valid = (iota + o2) < rem
      rg = jnp.where(valid, rowc[pl.ds(o2, L)], 0)
      lcg = jnp.where(valid, colc[pl.ds(o2, L)], 0)
      idxg16[0] = rg
      pltpu.sync_copy(x_hbm.at[idxg16.at[0]], gb.at[pl.ds(0, L)])
      for j in range(L):
        av = plsc.load_gather(attrc, [jnp.full((L,), o2 + j, jnp.int32)])
        av = av * jnp.where(o2 + j < rem, 1.0, 0.0)
        for r in range(DG):
          gb[j, pl.ds(r * L, L)] = gb[j, pl.ds(r * L, L)] * av
      idxs16[0] = lcg
      pltpu.sync_copy(gb.at[pl.ds(0, L)], agg_sh.at[idxs16.at[0]], add=True)
      return 0
    lax.fori_loop(0, g2n, p2f, 0)

    plsc.subcore_barrier()

    def sel_cp(k, off):
      lo = pl.multiple_of(nb0 + k * L, 8)
      ids = lo + iota
      m = (selw_v[pl.ds(lo, L)] > 0.5) & ((base_n + ids) < N)
      plsc.store_compressed(lsel.at[pl.ds(off, L)], ids, mask=m)
      return off + jnp.max(plsc.all_reduce_population_count(m))
    scnt = lax.fori_loop(0, KG, sel_cp, jnp.int32(0))

    def zd(k, _):
      st = pl.multiple_of(base_n, 8) + pl.multiple_of(nb0 + k * L, 8)
      @pl.when(st < N)
      def _():
        pltpu.make_async_copy(xb, out_hbm.at[pl.ds(st, L)], osem).wait()
      return 0
    lax.fori_loop(0, KG, zd, 0)

    g3n = (scnt + (L - 1)) // L

    def p3(g3, _):
      o3 = pl.multiple_of(g3 * L, L)
      valid = (iota + o3) < scnt
      last = plsc.load_gather(lsel, [jnp.full((L,), scnt - 1, jnp.int32)])
      ids = jnp.where(valid, lsel[pl.ds(o3, L)], last)
      gids = ids + base_n
      idxg16[0] = gids
      idxs16[0] = ids
      pltpu.async_copy(x_hbm.at[idxg16.at[0]], xb, rsem)
      pltpu.async_copy(agg_sh.at[idxs16.at[0]], ab, csem)
      pltpu.make_async_copy(x_hbm.at[idxg16.at[0]], xb, rsem).wait()
      pltpu.make_async_copy(agg_sh.at[idxs16.at[0]], ab, csem).wait()
      for j in range(L):
        for r in range(DG):
          xb[j, pl.ds(r * L, L)] = (
              xb[j, pl.ds(r * L, L)] + ab[j, pl.ds(r * L, L)])
      pltpu.sync_copy(xb, out_hbm.at[idxg16.at[0]])
      return 0
    lax.fori_loop(0, g3n, p3, 0)

  mesh = plsc.VectorSubcoreMesh(
      core_axis_name="c", subcore_axis_name="s", num_cores=NC, num_subcores=NS)
  run = pl.kernel(
      body,
      out_type=jax.ShapeDtypeStruct((N, D), jnp.float32),
      mesh=mesh,
      compiler_params=pltpu.CompilerParams(needs_layout_passes=False),
      scratch_types=[
          pltpu.VMEM((N,), jnp.float32),
          pltpu.VMEM((NP,), jnp.float32),
          pltpu.VMEM((2 * C,), jnp.int32),
          pltpu.VMEM((2 * C,), jnp.int32),
          pltpu.VMEM((2 * C,), jnp.float32),
          pltpu.VMEM((C + BE,), jnp.int32),
          pltpu.VMEM((C + BE,), jnp.int32),
          pltpu.VMEM((C + BE,), jnp.float32),
          pltpu.VMEM((H,), jnp.float32),
          pltpu.VMEM((NS * TN,), jnp.float32),
          pltpu.VMEM((H,), jnp.float32),
          pltpu.VMEM((TN,), jnp.int32),
          pltpu.VMEM((BE, D), jnp.float32),
          pltpu.VMEM((L, D), jnp.float32),
          pltpu.VMEM((L, D), jnp.float32),
          pltpu.VMEM((1, BE), jnp.int32),
          pltpu.VMEM((1, BE), jnp.int32),
          pltpu.VMEM((1, L), jnp.int32),
          pltpu.VMEM((1, L), jnp.int32),
          pltpu.VMEM((TN,), jnp.float32),
          pltpu.SemaphoreType.DMA,
          pltpu.SemaphoreType.DMA,
          pltpu.SemaphoreType.DMA,
          pltpu.SemaphoreType.DMA,
          pltpu.SemaphoreType.DMA,
          pltpu.SemaphoreType.DMA,
          pltpu.SemaphoreType.DMA,
          pltpu.SemaphoreType.DMA,
          pltpu.VMEM_SHARED((NS * H,), jnp.float32),
          pltpu.VMEM_SHARED((H,), jnp.float32),
          pltpu.VMEM_SHARED((H, D), jnp.float32),
      ],
  )
  return run(x, row, col, attr, order)


@jax.jit
def kernel(x, edge_index, edge_attr, batch):
  N, D = x.shape
  E = edge_index.shape[1]
  assert D % L == 0 and N % L == 0 and E % (NS * L) == 0
  order = jax.random.uniform(jax.random.key(42), (N,), dtype=jnp.float32)
  return _sc_pool(x, edge_index[0], edge_index[1], edge_attr, order,
                  N=N, D=D, E=E)


# speedup vs baseline: 1.9859x; 1.0025x over previous
import functools

import jax
import jax.numpy as jnp
from jax import lax
from jax.experimental import pallas as pl
from jax.experimental.pallas import tpu as pltpu
from jax.experimental.pallas import tpu_sc as plsc

L = 16
NC = 2
NS = 16
C = 2048
BE = 64


def _sc_pool(x, row, col, attr, order, *, N, D, E):
  H = ((N + NC * NS * 128 - 1) // (NC * NS * 128)) * NS * 128
  NP = NC * H
  EC = E // NS
  NCHF = EC // C
  CL = EC - NCHF * C
  NCH = NCHF + (1 if CL else 0)
  TN = H // NS
  KG = TN // L
  DG = D // L

  def clen(ci):
    return C if ci < NCHF else CL

  def body(x_hbm, row_hbm, col_hbm, attr_hbm, order_hbm, out_hbm,
           order_v, cnt_v, rowb, colb, attrb, rowc, colc, attrc,
           selw_v, cnt2, lsel, gb, xb, ab, idxg, idxs,
           idxg16, idxs16, red, rsem, csem, asem, zsem, osem,
           trsem, tcsem, tasem,
           stag_sh, selcnt_sh, agg_sh):
    c = lax.axis_index("c")
    s = lax.axis_index("s")
    base_n = c * H
    e0 = pl.multiple_of(s * EC, 8)
    nb0 = pl.multiple_of(s * TN, 128)

    zeros = jnp.zeros((L,), jnp.float32)
    ones = jnp.ones((L,), jnp.float32)
    iota = lax.iota(jnp.int32, L)

    pltpu.async_copy(order_hbm, order_v, rsem)
    for j in range(L):
      for r in range(DG):
        gb[j, pl.ds(r * L, L)] = zeros
        xb[j, pl.ds(r * L, L)] = zeros
    for k in range(KG):
      pltpu.async_copy(
          gb.at[pl.ds(0, L)],
          agg_sh.at[pl.ds(pl.multiple_of(nb0 + k * L, 8), L)], zsem)

    def zf(k, _):
      st = pl.multiple_of(base_n, 8) + pl.multiple_of(nb0 + k * L, 8)
      @pl.when(st < N)
      def _():
        pltpu.async_copy(xb, out_hbm.at[pl.ds(st, L)], osem)
      return 0
    lax.fori_loop(0, KG, zf, 0)

    def zc(i, _):
      o = pl.multiple_of(i * (L * 8), L)
      for u in range(8):
        cnt_v[pl.ds(o + u * L, L)] = zeros
        selw_v[pl.ds(o + u * L, L)] = zeros
      return 0
    lax.fori_loop(0, NP // (L * 8), zc, 0)
    pltpu.make_async_copy(order_hbm, order_v, rsem).wait()

    pltpu.async_copy(row_hbm.at[pl.ds(e0, C)], rowb.at[pl.ds(0, C)], rsem)
    pltpu.async_copy(col_hbm.at[pl.ds(e0, C)], colb.at[pl.ds(0, C)], csem)

    def p1_body(b):
      def go(g):
        o = pl.multiple_of(b + g * L, L)
        rg = rowb[pl.ds(o, L)]
        cg = colb[pl.ds(o, L)]
        orv = plsc.load_gather(order_v, [rg])
        ocv = plsc.load_gather(order_v, [cg])
        fc = jnp.where(orv <= ocv, ones, zeros)
        fr = jnp.where(ocv <= orv, ones, zeros)
        plsc.addupdate_scatter(cnt_v, [cg], fc)
        plsc.addupdate_scatter(cnt_v, [rg], fr)
      return go

    if CL:
      eot = pl.multiple_of(e0 + NCHF * C, 8)
      pltpu.async_copy(row_hbm.at[pl.ds(eot, CL)], rowb.at[pl.ds(C, CL)],
                       trsem)
      pltpu.async_copy(col_hbm.at[pl.ds(eot, CL)], colb.at[pl.ds(C, CL)],
                       tcsem)
      pltpu.make_async_copy(
          row_hbm.at[pl.ds(eot, CL)], rowb.at[pl.ds(C, CL)], trsem).wait()
      pltpu.make_async_copy(
          col_hbm.at[pl.ds(eot, CL)], colb.at[pl.ds(C, CL)], tcsem).wait()
      plsc.parallel_loop(0, CL // L, unroll=8)(p1_body(C))

    def chunk1(ci, _):
      b = pl.multiple_of(lax.rem(ci, 2) * C, 128)
      eo = pl.multiple_of(e0 + ci * C, 8)
      pltpu.make_async_copy(
          row_hbm.at[pl.ds(eo, C)], rowb.at[pl.ds(b, C)], rsem).wait()
      pltpu.make_async_copy(
          col_hbm.at[pl.ds(eo, C)], colb.at[pl.ds(b, C)], csem).wait()

      @pl.when(ci + 1 < NCHF)
      def _():
        nb = pl.multiple_of(lax.rem(ci + 1, 2) * C, 128)
        no = pl.multiple_of(e0 + (ci + 1) * C, 8)
        pltpu.async_copy(row_hbm.at[pl.ds(no, C)], rowb.at[pl.ds(nb, C)],
                         rsem)
        pltpu.async_copy(col_hbm.at[pl.ds(no, C)], colb.at[pl.ds(nb, C)],
                         csem)
      plsc.parallel_loop(0, C // L, unroll=8)(p1_body(b))
      return 0
    lax.fori_loop(0, NCHF, chunk1, 0)

    pltpu.sync_copy(cnt_v.at[pl.ds(pl.multiple_of(base_n, 128), H)],
                    stag_sh.at[pl.ds(pl.multiple_of(s * H, 128), H)])
    plsc.subcore_barrier()
    for t in range(NS):
      pltpu.async_copy(
          stag_sh.at[pl.ds(pl.multiple_of(t * H + nb0, 128), TN)],
          cnt2.at[pl.ds(t * TN, TN)], rsem)
    for t in range(NS):
      pltpu.make_async_copy(
          stag_sh.at[pl.ds(pl.multiple_of(t * H + nb0, 128), TN)],
          cnt2.at[pl.ds(t * TN, TN)], rsem).wait()
    def rd(k, _):
      o = pl.multiple_of(k * L, L)
      acc = cnt2[pl.ds(o, L)]
      for t in range(1, NS):
        acc = acc + cnt2[pl.ds(t * TN + o, L)]
      red[pl.ds(o, L)] = acc
      return 0
    lax.fori_loop(0, KG, rd, 0)
    pltpu.sync_copy(red, selcnt_sh.at[pl.ds(nb0, TN)])
    plsc.subcore_barrier()

    pltpu.sync_copy(selcnt_sh, selw_v.at[pl.ds(pl.multiple_of(base_n, 128), H)])
    def sw(k, _):
      o = pl.multiple_of(base_n, 128) + pl.multiple_of(k * (L * 4), L)
      for u in range(4):
        ou = o + u * L
        selw_v[pl.ds(ou, L)] = jnp.where(selw_v[pl.ds(ou, L)] == 0.0,
                                         ones, zeros)
      return 0
    lax.fori_loop(0, H // (L * 4), sw, 0)

    for k in range(KG):
      pltpu.make_async_copy(
          gb.at[pl.ds(0, L)],
          agg_sh.at[pl.ds(pl.multiple_of(nb0 + k * L, 8), L)], zsem).wait()
    plsc.subcore_barrier()

    pltpu.async_copy(row_hbm.at[pl.ds(e0, C)], rowb.at[pl.ds(0, C)], rsem)
    pltpu.async_copy(col_hbm.at[pl.ds(e0, C)], colb.at[pl.ds(0, C)], csem)
    pltpu.async_copy(attr_hbm.at[pl.ds(e0, C)], attrb.at[pl.ds(0, C)], asem)
    if CL:
      eot0 = pl.multiple_of(e0 + NCHF * C, 8)
      pltpu.async_copy(row_hbm.at[pl.ds(eot0, CL)], rowb.at[pl.ds(C, CL)],
                       trsem)
      pltpu.async_copy(col_hbm.at[pl.ds(eot0, CL)], colb.at[pl.ds(C, CL)],
                       tcsem)
      pltpu.async_copy(attr_hbm.at[pl.ds(eot0, CL)], attrb.at[pl.ds(C, CL)],
                       tasem)

    def cp_body(b):
      def go(g, off):
        o = pl.multiple_of(b + g * L, L)
        rg = rowb[pl.ds(o, L)]
        cg = colb[pl.ds(o, L)]
        ag = attrb[pl.ds(o, L)]
        keep = plsc.load_gather(selw_v, [cg]) > 0.5
        plsc.store_compressed(rowc.at[pl.ds(off, L)], rg, mask=keep)
        plsc.store_compressed(colc.at[pl.ds(off, L)], cg, mask=keep)
        plsc.store_compressed(attrc.at[pl.ds(off, L)], ag, mask=keep)
        return off + jnp.max(plsc.all_reduce_population_count(keep))
      return go

    def batch(g4, _):
      o4 = pl.multiple_of(g4 * BE, L)
      for k in range(BE // L):
        idxg[0, pl.ds(k * L, L)] = rowc[pl.ds(o4 + k * L, L)]
        idxs[0, pl.ds(k * L, L)] = colc[pl.ds(o4 + k * L, L)] - base_n
      pltpu.sync_copy(x_hbm.at[idxg.at[0]], gb)
      def scale(j, _):
        av = plsc.load_gather(attrc, [jnp.full((L,), o4 + j, jnp.int32)])
        for r in range(DG):
          gb[j, pl.ds(r * L, L)] = gb[j, pl.ds(r * L, L)] * av
        return 0
      lax.fori_loop(0, BE, scale, 0)
      pltpu.sync_copy(gb, agg_sh.at[idxs.at[0]], add=True)
      return 0

    def process(kept):
      nfull = kept // BE
      lax.fori_loop(0, nfull, batch, 0)
      rem = kept - nfull * BE
      for k in range(BE // L):
        @pl.when(k * L < rem)
        def _():
          src = pl.multiple_of(nfull * BE, L) + k * L
          rowc[pl.ds(k * L, L)] = rowc[pl.ds(src, L)]
          colc[pl.ds(k * L, L)] = colc[pl.ds(src, L)]
          attrc[pl.ds(k * L, L)] = attrc[pl.ds(src, L)]
      return rem

    rem0 = jnp.int32(0)
    if CL:
      eot = pl.multiple_of(e0 + NCHF * C, 8)
      pltpu.make_async_copy(
          row_hbm.at[pl.ds(eot, CL)], rowb.at[pl.ds(C, CL)], trsem).wait()
      pltpu.make_async_copy(
          col_hbm.at[pl.ds(eot, CL)], colb.at[pl.ds(C, CL)], tcsem).wait()
      pltpu.make_async_copy(
          attr_hbm.at[pl.ds(eot, CL)], attrb.at[pl.ds(C, CL)], tasem).wait()
      kt = plsc.parallel_loop(0, CL // L, unroll=8, carry=rem0)(cp_body(C))
      rem0 = process(kt)

    def chunk2(ci, rem):
      b = pl.multiple_of(lax.rem(ci, 2) * C, 128)
      eo = pl.multiple_of(e0 + ci * C, 8)
      pltpu.make_async_copy(
          row_hbm.at[pl.ds(eo, C)], rowb.at[pl.ds(b, C)], rsem).wait()
      pltpu.make_async_copy(
          col_hbm.at[pl.ds(eo, C)], colb.at[pl.ds(b, C)], csem).wait()
      pltpu.make_async_copy(
          attr_hbm.at[pl.ds(eo, C)], attrb.at[pl.ds(b, C)], asem).wait()

      @pl.when(ci + 1 < NCHF)
      def _():
        nb = pl.multiple_of(lax.rem(ci + 1, 2) * C, 128)
        no = pl.multiple_of(e0 + (ci + 1) * C, 8)
        pltpu.async_copy(row_hbm.at[pl.ds(no, C)], rowb.at[pl.ds(nb, C)],
                         rsem)
        pltpu.async_copy(col_hbm.at[pl.ds(no, C)], colb.at[pl.ds(nb, C)],
                         csem)
        pltpu.async_copy(attr_hbm.at[pl.ds(no, C)], attrb.at[pl.ds(nb, C)],
                         asem)
      kept = plsc.parallel_loop(0, C // L, unroll=8, carry=rem)(cp_body(b))
      return process(kept)
    rem = lax.fori_loop(0, NCHF, chunk2, rem0)

    g2n = (rem + (L - 1)) // L

    def p2f(g2, _):
      o2 = pl.multiple_of(g2 * L, L)
      valid = (iota + o2) < rem
      rg = jnp.where(valid, rowc[pl.ds(o2, L)], 0)
      lcg = jnp.where(valid, colc[pl.ds(o2, L)] - base_n, 0)
      idxg16[0] = rg
      pltpu.sync_copy(x_hbm.at[idxg16.at[0]], gb.at[pl.ds(0, L)])
      for j in range(L):
        av = plsc.load_gather(attrc, [jnp.full((L,), o2 + j, jnp.int32)])
        av = av * jnp.where(o2 + j < rem, 1.0, 0.0)
        for r in range(DG):
          gb[j, pl.ds(r * L, L)] = gb[j, pl.ds(r * L, L)] * av
      idxs16[0] = lcg
      pltpu.sync_copy(gb.at[pl.ds(0, L)], agg_sh.at[idxs16.at[0]], add=True)
      return 0
    lax.fori_loop(0, g2n, p2f, 0)

    plsc.subcore_barrier()

    def sel_cp(k, off):
      lo = pl.multiple_of(nb0 + k * L, 8)
      ids = lo + iota
      glo = pl.multiple_of(base_n, 128) + lo
      m = (selw_v[pl.ds(glo, L)] > 0.5) & ((base_n + ids) < N)
      plsc.store_compressed(lsel.at[pl.ds(off, L)], ids, mask=m)
      return off + jnp.max(plsc.all_reduce_population_count(m))
    scnt = lax.fori_loop(0, KG, sel_cp, jnp.int32(0))

    def zd(k, _):
      st = pl.multiple_of(base_n, 8) + pl.multiple_of(nb0 + k * L, 8)
      @pl.when(st < N)
      def _():
        pltpu.make_async_copy(xb, out_hbm.at[pl.ds(st, L)], osem).wait()
      return 0
    lax.fori_loop(0, KG, zd, 0)

    g3n = (scnt + (L - 1)) // L

    def p3(g3, _):
      o3 = pl.multiple_of(g3 * L, L)
      valid = (iota + o3) < scnt
      last = plsc.load_gather(lsel, [jnp.full((L,), scnt - 1, jnp.int32)])
      ids = jnp.where(valid, lsel[pl.ds(o3, L)], last)
      gids = ids + base_n
      idxg16[0] = gids
      idxs16[0] = ids
      pltpu.async_copy(x_hbm.at[idxg16.at[0]], xb, rsem)
      pltpu.async_copy(agg_sh.at[idxs16.at[0]], ab, csem)
      pltpu.make_async_copy(x_hbm.at[idxg16.at[0]], xb, rsem).wait()
      pltpu.make_async_copy(agg_sh.at[idxs16.at[0]], ab, csem).wait()
      for j in range(L):
        for r in range(DG):
          xb[j, pl.ds(r * L, L)] = (
              xb[j, pl.ds(r * L, L)] + ab[j, pl.ds(r * L, L)])
      pltpu.sync_copy(xb, out_hbm.at[idxg16.at[0]])
      return 0
    lax.fori_loop(0, g3n, p3, 0)

  mesh = plsc.VectorSubcoreMesh(
      core_axis_name="c", subcore_axis_name="s", num_cores=NC, num_subcores=NS)
  run = pl.kernel(
      body,
      out_type=jax.ShapeDtypeStruct((N, D), jnp.float32),
      mesh=mesh,
      compiler_params=pltpu.CompilerParams(needs_layout_passes=False),
      scratch_types=[
          pltpu.VMEM((N,), jnp.float32),
          pltpu.VMEM((NP,), jnp.float32),
          pltpu.VMEM((2 * C,), jnp.int32),
          pltpu.VMEM((2 * C,), jnp.int32),
          pltpu.VMEM((2 * C,), jnp.float32),
          pltpu.VMEM((C + BE,), jnp.int32),
          pltpu.VMEM((C + BE,), jnp.int32),
          pltpu.VMEM((C + BE,), jnp.float32),
          pltpu.VMEM((NP,), jnp.float32),
          pltpu.VMEM((NS * TN,), jnp.float32),
          pltpu.VMEM((TN,), jnp.int32),
          pltpu.VMEM((BE, D), jnp.float32),
          pltpu.VMEM((L, D), jnp.float32),
          pltpu.VMEM((L, D), jnp.float32),
          pltpu.VMEM((1, BE), jnp.int32),
          pltpu.VMEM((1, BE), jnp.int32),
          pltpu.VMEM((1, L), jnp.int32),
          pltpu.VMEM((1, L), jnp.int32),
          pltpu.VMEM((TN,), jnp.float32),
          pltpu.SemaphoreType.DMA,
          pltpu.SemaphoreType.DMA,
          pltpu.SemaphoreType.DMA,
          pltpu.SemaphoreType.DMA,
          pltpu.SemaphoreType.DMA,
          pltpu.SemaphoreType.DMA,
          pltpu.SemaphoreType.DMA,
          pltpu.SemaphoreType.DMA,
          pltpu.VMEM_SHARED((NS * H,), jnp.float32),
          pltpu.VMEM_SHARED((H,), jnp.float32),
          pltpu.VMEM_SHARED((H, D), jnp.float32),
      ],
  )
  return run(x, row, col, attr, order)


@jax.jit
def kernel(x, edge_index, edge_attr, batch):
  N, D = x.shape
  E = edge_index.shape[1]
  assert D % L == 0 and N % L == 0 and E % (NS * L) == 0
  order = jax.random.uniform(jax.random.key(42), (N,), dtype=jnp.float32)
  return _sc_pool(x, edge_index[0], edge_index[1], edge_attr, order,
                  N=N, D=D, E=E)


# speedup vs baseline: 1.9905x; 1.0023x over previous
import functools

import jax
import jax.numpy as jnp
from jax import lax
from jax.experimental import pallas as pl
from jax.experimental.pallas import tpu as pltpu
from jax.experimental.pallas import tpu_sc as plsc

L = 16
NC = 2
NS = 16
C = 2048
BE = 64


def _sc_pool(x, row, col, attr, order, *, N, D, E):
  H = ((N + NC * NS * 128 - 1) // (NC * NS * 128)) * NS * 128
  NP = NC * H
  EC = E // NS
  NCHF = EC // C
  CL = EC - NCHF * C
  NCH = NCHF + (1 if CL else 0)
  TN = H // NS
  KG = TN // L
  DG = D // L

  def clen(ci):
    return C if ci < NCHF else CL

  def body(x_hbm, row_hbm, col_hbm, attr_hbm, order_hbm, out_hbm,
           order_v, cnt_v, rowb, colb, attrb, rowc, colc, attrc,
           selw_v, cnt2, lsel, gb, xb, ab, idxg, idxs,
           idxg16, idxs16, red, rsem, csem, asem, zsem, osem,
           trsem, tcsem, tasem,
           stag_sh, selcnt_sh, agg_sh):
    c = lax.axis_index("c")
    s = lax.axis_index("s")
    base_n = c * H
    e0 = pl.multiple_of(s * EC, 8)
    nb0 = pl.multiple_of(s * TN, 128)

    zeros = jnp.zeros((L,), jnp.float32)
    ones = jnp.ones((L,), jnp.float32)
    iota = lax.iota(jnp.int32, L)

    pltpu.async_copy(order_hbm, order_v, rsem)
    for j in range(L):
      for r in range(DG):
        gb[j, pl.ds(r * L, L)] = zeros
        xb[j, pl.ds(r * L, L)] = zeros
    for k in range(KG):
      pltpu.async_copy(
          gb.at[pl.ds(0, L)],
          agg_sh.at[pl.ds(pl.multiple_of(nb0 + k * L, 8), L)], zsem)

    def zf(k, _):
      st = pl.multiple_of(base_n, 8) + pl.multiple_of(nb0 + k * L, 8)
      @pl.when(st < N)
      def _():
        pltpu.async_copy(xb, out_hbm.at[pl.ds(st, L)], osem)
      return 0
    lax.fori_loop(0, KG, zf, 0)

    def zc(i, _):
      o = pl.multiple_of(i * (L * 8), L)
      for u in range(8):
        cnt_v[pl.ds(o + u * L, L)] = zeros
        selw_v[pl.ds(o + u * L, L)] = zeros
      return 0
    lax.fori_loop(0, NP // (L * 8), zc, 0)
    pltpu.make_async_copy(order_hbm, order_v, rsem).wait()

    _ns = jax.named_scope("ph_pass1"); _ns.__enter__()
    pltpu.async_copy(row_hbm.at[pl.ds(e0, C)], rowb.at[pl.ds(0, C)], rsem)
    pltpu.async_copy(col_hbm.at[pl.ds(e0, C)], colb.at[pl.ds(0, C)], csem)

    def p1_body(b):
      def go(g):
        o = pl.multiple_of(b + g * L, L)
        rg = rowb[pl.ds(o, L)]
        cg = colb[pl.ds(o, L)]
        orv = plsc.load_gather(order_v, [rg])
        ocv = plsc.load_gather(order_v, [cg])
        fc = jnp.where(orv <= ocv, ones, zeros)
        fr = jnp.where(ocv <= orv, ones, zeros)
        plsc.addupdate_scatter(cnt_v, [cg], fc)
        plsc.addupdate_scatter(cnt_v, [rg], fr)
      return go

    if CL:
      eot = pl.multiple_of(e0 + NCHF * C, 8)
      pltpu.async_copy(row_hbm.at[pl.ds(eot, CL)], rowb.at[pl.ds(C, CL)],
                       trsem)
      pltpu.async_copy(col_hbm.at[pl.ds(eot, CL)], colb.at[pl.ds(C, CL)],
                       tcsem)
      pltpu.make_async_copy(
          row_hbm.at[pl.ds(eot, CL)], rowb.at[pl.ds(C, CL)], trsem).wait()
      pltpu.make_async_copy(
          col_hbm.at[pl.ds(eot, CL)], colb.at[pl.ds(C, CL)], tcsem).wait()
      plsc.parallel_loop(0, CL // L, unroll=8)(p1_body(C))

    def chunk1(ci, _):
      b = pl.multiple_of(lax.rem(ci, 2) * C, 128)
      eo = pl.multiple_of(e0 + ci * C, 8)
      pltpu.make_async_copy(
          row_hbm.at[pl.ds(eo, C)], rowb.at[pl.ds(b, C)], rsem).wait()
      pltpu.make_async_copy(
          col_hbm.at[pl.ds(eo, C)], colb.at[pl.ds(b, C)], csem).wait()

      @pl.when(ci + 1 < NCHF)
      def _():
        nb = pl.multiple_of(lax.rem(ci + 1, 2) * C, 128)
        no = pl.multiple_of(e0 + (ci + 1) * C, 8)
        pltpu.async_copy(row_hbm.at[pl.ds(no, C)], rowb.at[pl.ds(nb, C)],
                         rsem)
        pltpu.async_copy(col_hbm.at[pl.ds(no, C)], colb.at[pl.ds(nb, C)],
                         csem)
      plsc.parallel_loop(0, C // L, unroll=8)(p1_body(b))
      return 0
    lax.fori_loop(0, NCHF, chunk1, 0)

    _ns.__exit__(None, None, None)
    _ns = jax.named_scope("ph_reduce"); _ns.__enter__()
    pltpu.sync_copy(cnt_v.at[pl.ds(pl.multiple_of(base_n, 128), H)],
                    stag_sh.at[pl.ds(pl.multiple_of(s * H, 128), H)])
    plsc.subcore_barrier()
    for t in range(NS):
      pltpu.async_copy(
          stag_sh.at[pl.ds(pl.multiple_of(t * H + nb0, 128), TN)],
          cnt2.at[pl.ds(t * TN, TN)], rsem)
    for t in range(NS):
      pltpu.make_async_copy(
          stag_sh.at[pl.ds(pl.multiple_of(t * H + nb0, 128), TN)],
          cnt2.at[pl.ds(t * TN, TN)], rsem).wait()
    def rd(k, _):
      o = pl.multiple_of(k * L, L)
      acc = cnt2[pl.ds(o, L)]
      for t in range(1, NS):
        acc = acc + cnt2[pl.ds(t * TN + o, L)]
      red[pl.ds(o, L)] = acc
      return 0
    lax.fori_loop(0, KG, rd, 0)
    pltpu.sync_copy(red, selcnt_sh.at[pl.ds(nb0, TN)])
    plsc.subcore_barrier()

    pltpu.sync_copy(selcnt_sh, selw_v.at[pl.ds(pl.multiple_of(base_n, 128), H)])
    def sw(k, _):
      o = pl.multiple_of(base_n, 128) + pl.multiple_of(k * (L * 4), L)
      for u in range(4):
        ou = o + u * L
        selw_v[pl.ds(ou, L)] = jnp.where(selw_v[pl.ds(ou, L)] == 0.0,
                                         ones, zeros)
      return 0
    lax.fori_loop(0, H // (L * 4), sw, 0)

    for k in range(KG):
      pltpu.make_async_copy(
          gb.at[pl.ds(0, L)],
          agg_sh.at[pl.ds(pl.multiple_of(nb0 + k * L, 8), L)], zsem).wait()
    plsc.subcore_barrier()

    _ns.__exit__(None, None, None)
    _ns = jax.named_scope("ph_pass2"); _ns.__enter__()
    pltpu.async_copy(row_hbm.at[pl.ds(e0, C)], rowb.at[pl.ds(0, C)], rsem)
    pltpu.async_copy(col_hbm.at[pl.ds(e0, C)], colb.at[pl.ds(0, C)], csem)
    pltpu.async_copy(attr_hbm.at[pl.ds(e0, C)], attrb.at[pl.ds(0, C)], asem)
    if CL:
      eot0 = pl.multiple_of(e0 + NCHF * C, 8)
      pltpu.async_copy(row_hbm.at[pl.ds(eot0, CL)], rowb.at[pl.ds(C, CL)],
                       trsem)
      pltpu.async_copy(col_hbm.at[pl.ds(eot0, CL)], colb.at[pl.ds(C, CL)],
                       tcsem)
      pltpu.async_copy(attr_hbm.at[pl.ds(eot0, CL)], attrb.at[pl.ds(C, CL)],
                       tasem)

    def cp_body(b):
      def go(g, off):
        o = pl.multiple_of(b + g * L, L)
        rg = rowb[pl.ds(o, L)]
        cg = colb[pl.ds(o, L)]
        ag = attrb[pl.ds(o, L)]
        keep = plsc.load_gather(selw_v, [cg]) > 0.5
        plsc.store_compressed(rowc.at[pl.ds(off, L)], rg, mask=keep)
        plsc.store_compressed(colc.at[pl.ds(off, L)], cg, mask=keep)
        plsc.store_compressed(attrc.at[pl.ds(off, L)], ag, mask=keep)
        return off + jnp.max(plsc.all_reduce_population_count(keep))
      return go

    def batch(g4, _):
      o4 = pl.multiple_of(g4 * BE, L)
      for k in range(BE // L):
        idxg[0, pl.ds(k * L, L)] = rowc[pl.ds(o4 + k * L, L)]
        idxs[0, pl.ds(k * L, L)] = colc[pl.ds(o4 + k * L, L)] - base_n
      pltpu.sync_copy(x_hbm.at[idxg.at[0]], gb)
      def scale(j, _):
        av = plsc.load_gather(attrc, [jnp.full((L,), o4 + j, jnp.int32)])
        for r in range(DG):
          gb[j, pl.ds(r * L, L)] = gb[j, pl.ds(r * L, L)] * av
        return 0
      lax.fori_loop(0, BE, scale, 0)
      pltpu.sync_copy(gb, agg_sh.at[idxs.at[0]], add=True)
      return 0

    def process(kept):
      nfull = kept // BE
      lax.fori_loop(0, nfull, batch, 0)
      rem = kept - nfull * BE
      for k in range(BE // L):
        @pl.when(k * L < rem)
        def _():
          src = pl.multiple_of(nfull * BE, L) + k * L
          rowc[pl.ds(k * L, L)] = rowc[pl.ds(src, L)]
          colc[pl.ds(k * L, L)] = colc[pl.ds(src, L)]
          attrc[pl.ds(k * L, L)] = attrc[pl.ds(src, L)]
      return rem

    rem0 = jnp.int32(0)
    if CL:
      eot = pl.multiple_of(e0 + NCHF * C, 8)
      pltpu.make_async_copy(
          row_hbm.at[pl.ds(eot, CL)], rowb.at[pl.ds(C, CL)], trsem).wait()
      pltpu.make_async_copy(
          col_hbm.at[pl.ds(eot, CL)], colb.at[pl.ds(C, CL)], tcsem).wait()
      pltpu.make_async_copy(
          attr_hbm.at[pl.ds(eot, CL)], attrb.at[pl.ds(C, CL)], tasem).wait()
      kt = plsc.parallel_loop(0, CL // L, unroll=8, carry=rem0)(cp_body(C))
      rem0 = process(kt)

    def chunk2(ci, rem):
      b = pl.multiple_of(lax.rem(ci, 2) * C, 128)
      eo = pl.multiple_of(e0 + ci * C, 8)
      pltpu.make_async_copy(
          row_hbm.at[pl.ds(eo, C)], rowb.at[pl.ds(b, C)], rsem).wait()
      pltpu.make_async_copy(
          col_hbm.at[pl.ds(eo, C)], colb.at[pl.ds(b, C)], csem).wait()
      pltpu.make_async_copy(
          attr_hbm.at[pl.ds(eo, C)], attrb.at[pl.ds(b, C)], asem).wait()

      @pl.when(ci + 1 < NCHF)
      def _():
        nb = pl.multiple_of(lax.rem(ci + 1, 2) * C, 128)
        no = pl.multiple_of(e0 + (ci + 1) * C, 8)
        pltpu.async_copy(row_hbm.at[pl.ds(no, C)], rowb.at[pl.ds(nb, C)],
                         rsem)
        pltpu.async_copy(col_hbm.at[pl.ds(no, C)], colb.at[pl.ds(nb, C)],
                         csem)
        pltpu.async_copy(attr_hbm.at[pl.ds(no, C)], attrb.at[pl.ds(nb, C)],
                         asem)
      kept = plsc.parallel_loop(0, C // L, unroll=8, carry=rem)(cp_body(b))
      return process(kept)
    rem = lax.fori_loop(0, NCHF, chunk2, rem0)

    g2n = (rem + (L - 1)) // L

    def p2f(g2, _):
      o2 = pl.multiple_of(g2 * L, L)
      valid = (iota + o2) < rem
      rg = jnp.where(valid, rowc[pl.ds(o2, L)], 0)
      lcg = jnp.where(valid, colc[pl.ds(o2, L)] - base_n, 0)
      idxg16[0] = rg
      pltpu.sync_copy(x_hbm.at[idxg16.at[0]], gb.at[pl.ds(0, L)])
      for j in range(L):
        av = plsc.load_gather(attrc, [jnp.full((L,), o2 + j, jnp.int32)])
        av = av * jnp.where(o2 + j < rem, 1.0, 0.0)
        for r in range(DG):
          gb[j, pl.ds(r * L, L)] = gb[j, pl.ds(r * L, L)] * av
      idxs16[0] = lcg
      pltpu.sync_copy(gb.at[pl.ds(0, L)], agg_sh.at[idxs16.at[0]], add=True)
      return 0
    lax.fori_loop(0, g2n, p2f, 0)

    _ns.__exit__(None, None, None)
    _ns = jax.named_scope("ph_epi"); _ns.__enter__()
    plsc.subcore_barrier()

    def sel_cp(k, off):
      lo = pl.multiple_of(nb0 + k * L, 8)
      ids = lo + iota
      glo = pl.multiple_of(base_n, 128) + lo
      m = (selw_v[pl.ds(glo, L)] > 0.5) & ((base_n + ids) < N)
      plsc.store_compressed(lsel.at[pl.ds(off, L)], ids, mask=m)
      return off + jnp.max(plsc.all_reduce_population_count(m))
    scnt = lax.fori_loop(0, KG, sel_cp, jnp.int32(0))

    def zd(k, _):
      st = pl.multiple_of(base_n, 8) + pl.multiple_of(nb0 + k * L, 8)
      @pl.when(st < N)
      def _():
        pltpu.make_async_copy(xb, out_hbm.at[pl.ds(st, L)], osem).wait()
      return 0
    lax.fori_loop(0, KG, zd, 0)

    g3n = (scnt + (L - 1)) // L

    def p3(g3, _):
      o3 = pl.multiple_of(g3 * L, L)
      valid = (iota + o3) < scnt
      last = plsc.load_gather(lsel, [jnp.full((L,), scnt - 1, jnp.int32)])
      ids = jnp.where(valid, lsel[pl.ds(o3, L)], last)
      gids = ids + base_n
      idxg16[0] = gids
      idxs16[0] = ids
      pltpu.async_copy(x_hbm.at[idxg16.at[0]], xb, rsem)
      pltpu.async_copy(agg_sh.at[idxs16.at[0]], ab, csem)
      pltpu.make_async_copy(x_hbm.at[idxg16.at[0]], xb, rsem).wait()
      pltpu.make_async_copy(agg_sh.at[idxs16.at[0]], ab, csem).wait()
      for j in range(L):
        for r in range(DG):
          xb[j, pl.ds(r * L, L)] = (
              xb[j, pl.ds(r * L, L)] + ab[j, pl.ds(r * L, L)])
      pltpu.sync_copy(xb, out_hbm.at[idxg16.at[0]])
      return 0
    lax.fori_loop(0, g3n, p3, 0)
    _ns.__exit__(None, None, None)

  mesh = plsc.VectorSubcoreMesh(
      core_axis_name="c", subcore_axis_name="s", num_cores=NC, num_subcores=NS)
  run = pl.kernel(
      body,
      out_type=jax.ShapeDtypeStruct((N, D), jnp.float32),
      mesh=mesh,
      compiler_params=pltpu.CompilerParams(needs_layout_passes=False),
      scratch_types=[
          pltpu.VMEM((N,), jnp.float32),
          pltpu.VMEM((NP,), jnp.float32),
          pltpu.VMEM((2 * C,), jnp.int32),
          pltpu.VMEM((2 * C,), jnp.int32),
          pltpu.VMEM((2 * C,), jnp.float32),
          pltpu.VMEM((C + BE,), jnp.int32),
          pltpu.VMEM((C + BE,), jnp.int32),
          pltpu.VMEM((C + BE,), jnp.float32),
          pltpu.VMEM((NP,), jnp.float32),
          pltpu.VMEM((NS * TN,), jnp.float32),
          pltpu.VMEM((TN,), jnp.int32),
          pltpu.VMEM((BE, D), jnp.float32),
          pltpu.VMEM((L, D), jnp.float32),
          pltpu.VMEM((L, D), jnp.float32),
          pltpu.VMEM((1, BE), jnp.int32),
          pltpu.VMEM((1, BE), jnp.int32),
          pltpu.VMEM((1, L), jnp.int32),
          pltpu.VMEM((1, L), jnp.int32),
          pltpu.VMEM((TN,), jnp.float32),
          pltpu.SemaphoreType.DMA,
          pltpu.SemaphoreType.DMA,
          pltpu.SemaphoreType.DMA,
          pltpu.SemaphoreType.DMA,
          pltpu.SemaphoreType.DMA,
          pltpu.SemaphoreType.DMA,
          pltpu.SemaphoreType.DMA,
          pltpu.SemaphoreType.DMA,
          pltpu.VMEM_SHARED((NS * H,), jnp.float32),
          pltpu.VMEM_SHARED((H,), jnp.float32),
          pltpu.VMEM_SHARED((H, D), jnp.float32),
      ],
  )
  return run(x, row, col, attr, order)


@jax.jit
def kernel(x, edge_index, edge_attr, batch):
  N, D = x.shape
  E = edge_index.shape[1]
  assert D % L == 0 and N % L == 0 and E % (NS * L) == 0
  order = jax.random.uniform(jax.random.key(42), (N,), dtype=jnp.float32)
  return _sc_pool(x, edge_index[0], edge_index[1], edge_attr, order,
                  N=N, D=D, E=E)


# speedup vs baseline: 1.9933x; 1.0014x over previous
import functools

import jax
import jax.numpy as jnp
from jax import lax
from jax.experimental import pallas as pl
from jax.experimental.pallas import tpu as pltpu
from jax.experimental.pallas import tpu_sc as plsc

L = 16
NC = 2
NS = 16
C = 2048
BE = 32


def _sc_pool(x, row, col, attr, order, *, N, D, E):
  H = ((N + NC * NS * 128 - 1) // (NC * NS * 128)) * NS * 128
  NP = NC * H
  EC = E // NS
  NCHF = EC // C
  CL = EC - NCHF * C
  NCH = NCHF + (1 if CL else 0)
  TN = H // NS
  KG = TN // L
  DG = D // L

  def clen(ci):
    return C if ci < NCHF else CL

  def body(x_hbm, row_hbm, col_hbm, attr_hbm, order_hbm, out_hbm,
           order_v, cnt_v, rowb, colb, attrb, rowc, colc, attrc,
           selw_v, cnt2, lsel, gb, xb, ab, idxga, idxsa, idxgb, idxsb,
           idxg16, idxs16, red, rsem, csem, asem, zsem, osem,
           trsem, tcsem, tasem, gsem, ssem,
           stag_sh, selcnt_sh, agg_sh):
    c = lax.axis_index("c")
    s = lax.axis_index("s")
    base_n = c * H
    e0 = pl.multiple_of(s * EC, 8)
    nb0 = pl.multiple_of(s * TN, 128)

    zeros = jnp.zeros((L,), jnp.float32)
    ones = jnp.ones((L,), jnp.float32)
    iota = lax.iota(jnp.int32, L)

    pltpu.async_copy(order_hbm, order_v, rsem)
    for j in range(L):
      for r in range(DG):
        gb[j, pl.ds(r * L, L)] = zeros
        xb[j, pl.ds(r * L, L)] = zeros
    for k in range(KG):
      pltpu.async_copy(
          gb.at[pl.ds(0, L)],
          agg_sh.at[pl.ds(pl.multiple_of(nb0 + k * L, 8), L)], zsem)

    def zf(k, _):
      st = pl.multiple_of(base_n, 8) + pl.multiple_of(nb0 + k * L, 8)
      @pl.when(st < N)
      def _():
        pltpu.async_copy(xb, out_hbm.at[pl.ds(st, L)], osem)
      return 0
    lax.fori_loop(0, KG, zf, 0)

    def zc(i, _):
      o = pl.multiple_of(i * (L * 8), L)
      for u in range(8):
        cnt_v[pl.ds(o + u * L, L)] = zeros
        selw_v[pl.ds(o + u * L, L)] = zeros
      return 0
    lax.fori_loop(0, NP // (L * 8), zc, 0)
    pltpu.make_async_copy(order_hbm, order_v, rsem).wait()

    _ns = jax.named_scope("ph_pass1"); _ns.__enter__()
    pltpu.async_copy(row_hbm.at[pl.ds(e0, C)], rowb.at[pl.ds(0, C)], rsem)
    pltpu.async_copy(col_hbm.at[pl.ds(e0, C)], colb.at[pl.ds(0, C)], csem)

    def p1_body(b):
      def go(g):
        o = pl.multiple_of(b + g * L, L)
        rg = rowb[pl.ds(o, L)]
        cg = colb[pl.ds(o, L)]
        orv = plsc.load_gather(order_v, [rg])
        ocv = plsc.load_gather(order_v, [cg])
        fc = jnp.where(orv <= ocv, ones, zeros)
        fr = jnp.where(ocv <= orv, ones, zeros)
        plsc.addupdate_scatter(cnt_v, [cg], fc)
        plsc.addupdate_scatter(cnt_v, [rg], fr)
      return go

    if CL:
      eot = pl.multiple_of(e0 + NCHF * C, 8)
      pltpu.async_copy(row_hbm.at[pl.ds(eot, CL)], rowb.at[pl.ds(C, CL)],
                       trsem)
      pltpu.async_copy(col_hbm.at[pl.ds(eot, CL)], colb.at[pl.ds(C, CL)],
                       tcsem)
      pltpu.make_async_copy(
          row_hbm.at[pl.ds(eot, CL)], rowb.at[pl.ds(C, CL)], trsem).wait()
      pltpu.make_async_copy(
          col_hbm.at[pl.ds(eot, CL)], colb.at[pl.ds(C, CL)], tcsem).wait()
      plsc.parallel_loop(0, CL // L, unroll=8)(p1_body(C))

    def chunk1(ci, _):
      b = pl.multiple_of(lax.rem(ci, 2) * C, 128)
      eo = pl.multiple_of(e0 + ci * C, 8)
      pltpu.make_async_copy(
          row_hbm.at[pl.ds(eo, C)], rowb.at[pl.ds(b, C)], rsem).wait()
      pltpu.make_async_copy(
          col_hbm.at[pl.ds(eo, C)], colb.at[pl.ds(b, C)], csem).wait()

      @pl.when(ci + 1 < NCHF)
      def _():
        nb = pl.multiple_of(lax.rem(ci + 1, 2) * C, 128)
        no = pl.multiple_of(e0 + (ci + 1) * C, 8)
        pltpu.async_copy(row_hbm.at[pl.ds(no, C)], rowb.at[pl.ds(nb, C)],
                         rsem)
        pltpu.async_copy(col_hbm.at[pl.ds(no, C)], colb.at[pl.ds(nb, C)],
                         csem)
      plsc.parallel_loop(0, C // L, unroll=8)(p1_body(b))
      return 0
    lax.fori_loop(0, NCHF, chunk1, 0)

    _ns.__exit__(None, None, None)
    _ns = jax.named_scope("ph_reduce"); _ns.__enter__()
    pltpu.sync_copy(cnt_v.at[pl.ds(pl.multiple_of(base_n, 128), H)],
                    stag_sh.at[pl.ds(pl.multiple_of(s * H, 128), H)])
    plsc.subcore_barrier()
    for t in range(NS):
      pltpu.async_copy(
          stag_sh.at[pl.ds(pl.multiple_of(t * H + nb0, 128), TN)],
          cnt2.at[pl.ds(t * TN, TN)], rsem)
    for t in range(NS):
      pltpu.make_async_copy(
          stag_sh.at[pl.ds(pl.multiple_of(t * H + nb0, 128), TN)],
          cnt2.at[pl.ds(t * TN, TN)], rsem).wait()
    def rd(k, _):
      o = pl.multiple_of(k * L, L)
      acc = cnt2[pl.ds(o, L)]
      for t in range(1, NS):
        acc = acc + cnt2[pl.ds(t * TN + o, L)]
      red[pl.ds(o, L)] = acc
      return 0
    lax.fori_loop(0, KG, rd, 0)
    pltpu.sync_copy(red, selcnt_sh.at[pl.ds(nb0, TN)])
    plsc.subcore_barrier()

    pltpu.sync_copy(selcnt_sh, selw_v.at[pl.ds(pl.multiple_of(base_n, 128), H)])
    def sw(k, _):
      o = pl.multiple_of(base_n, 128) + pl.multiple_of(k * (L * 4), L)
      for u in range(4):
        ou = o + u * L
        selw_v[pl.ds(ou, L)] = jnp.where(selw_v[pl.ds(ou, L)] == 0.0,
                                         ones, zeros)
      return 0
    lax.fori_loop(0, H // (L * 4), sw, 0)

    for k in range(KG):
      pltpu.make_async_copy(
          gb.at[pl.ds(0, L)],
          agg_sh.at[pl.ds(pl.multiple_of(nb0 + k * L, 8), L)], zsem).wait()
    plsc.subcore_barrier()

    _ns.__exit__(None, None, None)
    _ns = jax.named_scope("ph_pass2"); _ns.__enter__()
    pltpu.async_copy(row_hbm.at[pl.ds(e0, C)], rowb.at[pl.ds(0, C)], rsem)
    pltpu.async_copy(col_hbm.at[pl.ds(e0, C)], colb.at[pl.ds(0, C)], csem)
    pltpu.async_copy(attr_hbm.at[pl.ds(e0, C)], attrb.at[pl.ds(0, C)], asem)
    if CL:
      eot0 = pl.multiple_of(e0 + NCHF * C, 8)
      pltpu.async_copy(row_hbm.at[pl.ds(eot0, CL)], rowb.at[pl.ds(C, CL)],
                       trsem)
      pltpu.async_copy(col_hbm.at[pl.ds(eot0, CL)], colb.at[pl.ds(C, CL)],
                       tcsem)
      pltpu.async_copy(attr_hbm.at[pl.ds(eot0, CL)], attrb.at[pl.ds(C, CL)],
                       tasem)

    def cp_body(b):
      def go(g, off):
        o = pl.multiple_of(b + g * L, L)
        rg = rowb[pl.ds(o, L)]
        cg = colb[pl.ds(o, L)]
        ag = attrb[pl.ds(o, L)]
        keep = plsc.load_gather(selw_v, [cg]) > 0.5
        plsc.store_compressed(rowc.at[pl.ds(off, L)], rg, mask=keep)
        plsc.store_compressed(colc.at[pl.ds(off, L)], cg, mask=keep)
        plsc.store_compressed(attrc.at[pl.ds(off, L)], ag, mask=keep)
        return off + jnp.max(plsc.all_reduce_population_count(keep))
      return go

    def bidx(t, ig, isr):
      o4 = pl.multiple_of(t * BE, L)
      for k in range(BE // L):
        ig[0, pl.ds(k * L, L)] = rowc[pl.ds(o4 + k * L, L)]
        isr[0, pl.ds(k * L, L)] = colc[pl.ds(o4 + k * L, L)] - base_n

    def gbh(h):
      return gb.at[pl.ds(h * BE, BE)]

    def fire_g(ig, h):
      pltpu.async_copy(x_hbm.at[ig.at[0]], gbh(h), gsem)

    def wait_g(ig, h):
      pltpu.make_async_copy(x_hbm.at[ig.at[0]], gbh(h), gsem).wait()

    def fire_s(isr, h):
      pltpu.async_copy(gbh(h), agg_sh.at[isr.at[0]], ssem, add=True)

    def wait_s(isr, h):
      pltpu.make_async_copy(gbh(h), agg_sh.at[isr.at[0]], ssem).wait()

    def scale(t, hbase):
      def go(j, _):
        av = plsc.load_gather(attrc, [jnp.full((L,), t * BE + j, jnp.int32)])
        for r in range(DG):
          gb[hbase + j, pl.ds(r * L, L)] = gb[hbase + j, pl.ds(r * L, L)] * av
        return 0
      lax.fori_loop(0, BE, go, 0)

    def process(kept):
      nfull = kept // BE

      @pl.when(nfull > 0)
      def _():
        bidx(0, idxga, idxsa)
        fire_g(idxga, 0)

      def pair(i, _):
        t0 = 2 * i

        @pl.when(i >= 1)
        def _():
          wait_s(idxsb, 1)
        wait_g(idxga, 0)

        @pl.when(t0 + 1 < nfull)
        def _():
          bidx(t0 + 1, idxgb, idxsb)
          fire_g(idxgb, 1)
        scale(t0, 0)
        fire_s(idxsa, 0)

        @pl.when(t0 + 1 < nfull)
        def _():
          wait_g(idxgb, 1)
          wait_s(idxsa, 0)

          @pl.when(t0 + 2 < nfull)
          def _():
            bidx(t0 + 2, idxga, idxsa)
            fire_g(idxga, 0)
          scale(t0 + 1, BE)
          fire_s(idxsb, 1)
        return 0
      lax.fori_loop(0, (nfull + 1) // 2, pair, 0)

      @pl.when(nfull > 0)
      def _():
        h_odd = lax.rem(nfull, 2) == 0
        @pl.when(h_odd)
        def _():
          wait_s(idxsb, 1)
        @pl.when(jnp.logical_not(h_odd))
        def _():
          wait_s(idxsa, 0)

      rem = kept - nfull * BE
      for k in range(BE // L):
        @pl.when(k * L < rem)
        def _():
          src = pl.multiple_of(nfull * BE, L) + k * L
          rowc[pl.ds(k * L, L)] = rowc[pl.ds(src, L)]
          colc[pl.ds(k * L, L)] = colc[pl.ds(src, L)]
          attrc[pl.ds(k * L, L)] = attrc[pl.ds(src, L)]
      return rem

    rem0 = jnp.int32(0)
    if CL:
      eot = pl.multiple_of(e0 + NCHF * C, 8)
      pltpu.make_async_copy(
          row_hbm.at[pl.ds(eot, CL)], rowb.at[pl.ds(C, CL)], trsem).wait()
      pltpu.make_async_copy(
          col_hbm.at[pl.ds(eot, CL)], colb.at[pl.ds(C, CL)], tcsem).wait()
      pltpu.make_async_copy(
          attr_hbm.at[pl.ds(eot, CL)], attrb.at[pl.ds(C, CL)], tasem).wait()
      kt = plsc.parallel_loop(0, CL // L, unroll=8, carry=rem0)(cp_body(C))
      rem0 = process(kt)

    def chunk2(ci, rem):
      b = pl.multiple_of(lax.rem(ci, 2) * C, 128)
      eo = pl.multiple_of(e0 + ci * C, 8)
      pltpu.make_async_copy(
          row_hbm.at[pl.ds(eo, C)], rowb.at[pl.ds(b, C)], rsem).wait()
      pltpu.make_async_copy(
          col_hbm.at[pl.ds(eo, C)], colb.at[pl.ds(b, C)], csem).wait()
      pltpu.make_async_copy(
          attr_hbm.at[pl.ds(eo, C)], attrb.at[pl.ds(b, C)], asem).wait()

      @pl.when(ci + 1 < NCHF)
      def _():
        nb = pl.multiple_of(lax.rem(ci + 1, 2) * C, 128)
        no = pl.multiple_of(e0 + (ci + 1) * C, 8)
        pltpu.async_copy(row_hbm.at[pl.ds(no, C)], rowb.at[pl.ds(nb, C)],
                         rsem)
        pltpu.async_copy(col_hbm.at[pl.ds(no, C)], colb.at[pl.ds(nb, C)],
                         csem)
        pltpu.async_copy(attr_hbm.at[pl.ds(no, C)], attrb.at[pl.ds(nb, C)],
                         asem)
      kept = plsc.parallel_loop(0, C // L, unroll=8, carry=rem)(cp_body(b))
      return process(kept)
    rem = lax.fori_loop(0, NCHF, chunk2, rem0)

    g2n = (rem + (L - 1)) // L

    def p2f(g2, _):
      o2 = pl.multiple_of(g2 * L, L)
      valid = (iota + o2) < rem
      rg = jnp.where(valid, rowc[pl.ds(o2, L)], 0)
      lcg = jnp.where(valid, colc[pl.ds(o2, L)] - base_n, 0)
      idxg16[0] = rg
      pltpu.sync_copy(x_hbm.at[idxg16.at[0]], gb.at[pl.ds(0, L)])
      for j in range(L):
        av = plsc.load_gather(attrc, [jnp.full((L,), o2 + j, jnp.int32)])
        av = av * jnp.where(o2 + j < rem, 1.0, 0.0)
        for r in range(DG):
          gb[j, pl.ds(r * L, L)] = gb[j, pl.ds(r * L, L)] * av
      idxs16[0] = lcg
      pltpu.sync_copy(gb.at[pl.ds(0, L)], agg_sh.at[idxs16.at[0]], add=True)
      return 0
    lax.fori_loop(0, g2n, p2f, 0)

    _ns.__exit__(None, None, None)
    _ns = jax.named_scope("ph_epi"); _ns.__enter__()
    plsc.subcore_barrier()

    def sel_cp(k, off):
      lo = pl.multiple_of(nb0 + k * L, 8)
      ids = lo + iota
      glo = pl.multiple_of(base_n, 128) + lo
      m = (selw_v[pl.ds(glo, L)] > 0.5) & ((base_n + ids) < N)
      plsc.store_compressed(lsel.at[pl.ds(off, L)], ids, mask=m)
      return off + jnp.max(plsc.all_reduce_population_count(m))
    scnt = lax.fori_loop(0, KG, sel_cp, jnp.int32(0))

    def zd(k, _):
      st = pl.multiple_of(base_n, 8) + pl.multiple_of(nb0 + k * L, 8)
      @pl.when(st < N)
      def _():
        pltpu.make_async_copy(xb, out_hbm.at[pl.ds(st, L)], osem).wait()
      return 0
    lax.fori_loop(0, KG, zd, 0)

    g3n = (scnt + (L - 1)) // L

    def p3(g3, _):
      o3 = pl.multiple_of(g3 * L, L)
      valid = (iota + o3) < scnt
      last = plsc.load_gather(lsel, [jnp.full((L,), scnt - 1, jnp.int32)])
      ids = jnp.where(valid, lsel[pl.ds(o3, L)], last)
      gids = ids + base_n
      idxg16[0] = gids
      idxs16[0] = ids
      pltpu.async_copy(x_hbm.at[idxg16.at[0]], xb, rsem)
      pltpu.async_copy(agg_sh.at[idxs16.at[0]], ab, csem)
      pltpu.make_async_copy(x_hbm.at[idxg16.at[0]], xb, rsem).wait()
      pltpu.make_async_copy(agg_sh.at[idxs16.at[0]], ab, csem).wait()
      for j in range(L):
        for r in range(DG):
          xb[j, pl.ds(r * L, L)] = (
              xb[j, pl.ds(r * L, L)] + ab[j, pl.ds(r * L, L)])
      pltpu.sync_copy(xb, out_hbm.at[idxg16.at[0]])
      return 0
    lax.fori_loop(0, g3n, p3, 0)
    _ns.__exit__(None, None, None)

  mesh = plsc.VectorSubcoreMesh(
      core_axis_name="c", subcore_axis_name="s", num_cores=NC, num_subcores=NS)
  run = pl.kernel(
      body,
      out_type=jax.ShapeDtypeStruct((N, D), jnp.float32),
      mesh=mesh,
      compiler_params=pltpu.CompilerParams(needs_layout_passes=False),
      scratch_types=[
          pltpu.VMEM((N,), jnp.float32),
          pltpu.VMEM((NP,), jnp.float32),
          pltpu.VMEM((2 * C,), jnp.int32),
          pltpu.VMEM((2 * C,), jnp.int32),
          pltpu.VMEM((2 * C,), jnp.float32),
          pltpu.VMEM((C + BE,), jnp.int32),
          pltpu.VMEM((C + BE,), jnp.int32),
          pltpu.VMEM((C + BE,), jnp.float32),
          pltpu.VMEM((NP,), jnp.float32),
          pltpu.VMEM((NS * TN,), jnp.float32),
          pltpu.VMEM((TN,), jnp.int32),
          pltpu.VMEM((2 * BE, D), jnp.float32),
          pltpu.VMEM((L, D), jnp.float32),
          pltpu.VMEM((L, D), jnp.float32),
          pltpu.VMEM((1, BE), jnp.int32),
          pltpu.VMEM((1, BE), jnp.int32),
          pltpu.VMEM((1, BE), jnp.int32),
          pltpu.VMEM((1, BE), jnp.int32),
          pltpu.VMEM((1, L), jnp.int32),
          pltpu.VMEM((1, L), jnp.int32),
          pltpu.VMEM((TN,), jnp.float32),
          pltpu.SemaphoreType.DMA,
          pltpu.SemaphoreType.DMA,
          pltpu.SemaphoreType.DMA,
          pltpu.SemaphoreType.DMA,
          pltpu.SemaphoreType.DMA,
          pltpu.SemaphoreType.DMA,
          pltpu.SemaphoreType.DMA,
          pltpu.SemaphoreType.DMA,
          pltpu.SemaphoreType.DMA,
          pltpu.SemaphoreType.DMA,
          pltpu.VMEM_SHARED((NS * H,), jnp.float32),
          pltpu.VMEM_SHARED((H,), jnp.float32),
          pltpu.VMEM_SHARED((H, D), jnp.float32),
      ],
  )
  return run(x, row, col, attr, order)


@jax.jit
def kernel(x, edge_index, edge_attr, batch):
  N, D = x.shape
  E = edge_index.shape[1]
  assert D % L == 0 and N % L == 0 and E % (NS * L) == 0
  order = jax.random.uniform(jax.random.key(42), (N,), dtype=jnp.float32)
  return _sc_pool(x, edge_index[0], edge_index[1], edge_attr, order,
                  N=N, D=D, E=E)
